# baseline (speedup 1.0000x reference)
import functools

import jax
import jax.numpy as jnp
from jax import lax
from jax.experimental import pallas as pl
from jax.experimental.pallas import tpu as pltpu
from jax.experimental.pallas import tpu_sc as plsc

D = 64
H = 32
NBLK = 1024
EBLK = 2048
CHUNK = 128
NW = 32
EPS = 1e-5

_SC_MESH = plsc.VectorSubcoreMesh(core_axis_name="c", subcore_axis_name="s")
_SC_PARAMS = pltpu.CompilerParams(use_tc_tiling_on_sc=False)


def _embed_body(x_ref, g_ref, b_ref, w1_ref, b1_ref, w2_ref, b2_ref, o_ref):
    x = x_ref[...]
    m = jnp.mean(x, axis=-1, keepdims=True)
    v = jnp.mean((x - m) ** 2, axis=-1, keepdims=True)
    xn = (x - m) * lax.rsqrt(v + EPS) * g_ref[...] + b_ref[...]
    h = jnp.maximum(xn @ w1_ref[...] + b1_ref[...], 0.0)
    o_ref[...] = jnp.maximum(h @ w2_ref[...] + b2_ref[...], 0.0)


def _embed(x, g, b, w1, b1, w2, b2):
    n, f = x.shape
    return pl.pallas_call(
        _embed_body,
        grid=(n // NBLK,),
        in_specs=[
            pl.BlockSpec((NBLK, f), lambda i: (i, 0)),
            pl.BlockSpec((1, f), lambda i: (0, 0)),
            pl.BlockSpec((1, f), lambda i: (0, 0)),
            pl.BlockSpec((f, D), lambda i: (0, 0)),
            pl.BlockSpec((1, D), lambda i: (0, 0)),
            pl.BlockSpec((D, D), lambda i: (0, 0)),
            pl.BlockSpec((1, D), lambda i: (0, 0)),
        ],
        out_specs=pl.BlockSpec((NBLK, D), lambda i: (i, 0)),
        out_shape=jax.ShapeDtypeStruct((n, D), jnp.float32),
    )(x, g[None, :], b[None, :], w1, b1[None, :], w2, b2[None, :])


def _pre_body(r_ref, l_ref, wl_ref, bl_ref, wr_ref, a_ref, b_ref):
    a_ref[...] = (r_ref[...] @ wl_ref[...] + bl_ref[...]).astype(jnp.bfloat16)
    b_ref[...] = (l_ref[...] @ wr_ref[...]).astype(jnp.bfloat16)


def _pre(right, left, wl, bl2, wr):
    n = right.shape[0]
    return pl.pallas_call(
        _pre_body,
        grid=(n // NBLK,),
        in_specs=[
            pl.BlockSpec((NBLK, D), lambda i: (i, 0)),
            pl.BlockSpec((NBLK, D), lambda i: (i, 0)),
            pl.BlockSpec((D, D), lambda i: (0, 0)),
            pl.BlockSpec((1, D), lambda i: (0, 0)),
            pl.BlockSpec((D, D), lambda i: (0, 0)),
        ],
        out_specs=[
            pl.BlockSpec((NBLK, D), lambda i: (i, 0)),
            pl.BlockSpec((NBLK, D), lambda i: (i, 0)),
        ],
        out_shape=[
            jax.ShapeDtypeStruct((n, D), jnp.bfloat16),
            jax.ShapeDtypeStruct((n, D), jnp.bfloat16),
        ],
    )(right, left, wl, bl2[None, :], wr)


def _edge_body(ma_ref, mb_ref, g_ref, b_ref, o_ref):
    m = ma_ref[...].astype(jnp.float32) + mb_ref[...].astype(jnp.float32)
    j = jnp.full((D, D), 1.0 / D, jnp.float32)
    mu = jax.lax.dot(m, j, precision=jax.lax.Precision.HIGHEST)
    cen = m - mu
    var = jax.lax.dot(cen * cen, j, precision=jax.lax.Precision.HIGHEST)
    u = jnp.maximum(cen * lax.rsqrt(var + EPS) * g_ref[...] + b_ref[...], 0.0)
    o_ref[0] = u[:, :H]
    o_ref[1] = u[:, H:]


def _edge(ma, mb, g, b):
    e = ma.shape[0]
    return pl.pallas_call(
        _edge_body,
        grid=(e // EBLK,),
        in_specs=[
            pl.BlockSpec((EBLK, D), lambda i: (i, 0)),
            pl.BlockSpec((EBLK, D), lambda i: (i, 0)),
            pl.BlockSpec((1, D), lambda i: (0, 0)),
            pl.BlockSpec((1, D), lambda i: (0, 0)),
        ],
        out_specs=pl.BlockSpec((2, EBLK, H), lambda i: (0, i, 0)),
        out_shape=jax.ShapeDtypeStruct((2, e, H), jnp.float32),
    )(ma, mb, g[None, :], b[None, :])


def _post_body(ulo_ref, uhi_ref, deg_ref, r_ref, wf_ref, bf_ref, lg_ref,
               lb_ref, wo1_ref, bo1_ref, wo2_ref, bo2_ref, o_ref):
    u = jnp.concatenate([ulo_ref[0], uhi_ref[0]], axis=-1)
    agg = u @ wf_ref[...] + deg_ref[...][:, :1] * bf_ref[...]
    mu = jnp.mean(agg, axis=-1, keepdims=True)
    var = jnp.mean((agg - mu) ** 2, axis=-1, keepdims=True)
    aggn = (agg - mu) * lax.rsqrt(var + EPS) * lg_ref[...] + lb_ref[...]
    cat = jnp.concatenate([aggn, r_ref[...]], axis=-1)
    h = jnp.maximum(cat @ wo1_ref[...] + bo1_ref[...], 0.0)
    o_ref[...] = h @ wo2_ref[...] + bo2_ref[...]


def _post(u2, deg, right, wf, bf, lg, lb, wo1, bo1, wo2, bo2):
    n = right.shape[0]
    return pl.pallas_call(
        _post_body,
        grid=(n // NBLK,),
        in_specs=[
            pl.BlockSpec((1, NBLK, H), lambda i: (0, i, 0)),
            pl.BlockSpec((1, NBLK, H), lambda i: (1, i, 0)),
            pl.BlockSpec((NBLK, H), lambda i: (i, 0)),
            pl.BlockSpec((NBLK, D), lambda i: (i, 0)),
            pl.BlockSpec((D, D), lambda i: (0, 0)),
            pl.BlockSpec((1, D), lambda i: (0, 0)),
            pl.BlockSpec((1, D), lambda i: (0, 0)),
            pl.BlockSpec((1, D), lambda i: (0, 0)),
            pl.BlockSpec((2 * D, D), lambda i: (0, 0)),
            pl.BlockSpec((1, D), lambda i: (0, 0)),
            pl.BlockSpec((D, D), lambda i: (0, 0)),
            pl.BlockSpec((1, D), lambda i: (0, 0)),
        ],
        out_specs=pl.BlockSpec((NBLK, D), lambda i: (i, 0)),
        out_shape=jax.ShapeDtypeStruct((n, D), jnp.float32),
    )(u2, u2, deg, right, wf, bf[None, :], lg[None, :], lb[None, :],
      wo1, bo1[None, :], wo2, bo2[None, :])


def _head_body(x_ref, w1_ref, b1_ref, w2_ref, o_ref):
    h = jnp.maximum(x_ref[...] @ w1_ref[...] + b1_ref[...], 0.0)
    o_ref[...] = h @ w2_ref[...]


def _head(x, w1, b1, w2p):
    n = x.shape[0]
    return pl.pallas_call(
        _head_body,
        grid=(n // NBLK,),
        in_specs=[
            pl.BlockSpec((NBLK, D), lambda i: (i, 0)),
            pl.BlockSpec((D, D), lambda i: (0, 0)),
            pl.BlockSpec((1, D), lambda i: (0, 0)),
            pl.BlockSpec((D, 128), lambda i: (0, 0)),
        ],
        out_specs=pl.BlockSpec((NBLK, 128), lambda i: (i, 0)),
        out_shape=jax.ShapeDtypeStruct((n, 128), jnp.float32),
    )(x, w1, b1[None, :], w2p)


def _make_gather(npad, epad):
    cpt = epad // (NW * CHUNK)

    @functools.partial(
        pl.kernel,
        mesh=_SC_MESH,
        compiler_params=_SC_PARAMS,
        out_type=[
            jax.ShapeDtypeStruct((epad, D), jnp.bfloat16),
            jax.ShapeDtypeStruct((epad, D), jnp.bfloat16),
        ],
        scratch_types=[
            pltpu.VMEM((cpt, CHUNK), jnp.int32),
            pltpu.VMEM((cpt, CHUNK), jnp.int32),
            pltpu.VMEM((CHUNK, D), jnp.bfloat16),
            pltpu.VMEM((CHUNK, D), jnp.bfloat16),
            pltpu.VMEM((CHUNK, D), jnp.bfloat16),
            pltpu.VMEM((CHUNK, D), jnp.bfloat16),
            pltpu.SemaphoreType.DMA,
            pltpu.SemaphoreType.DMA,
            pltpu.SemaphoreType.DMA,
            pltpu.SemaphoreType.DMA,
            pltpu.SemaphoreType.DMA,
            pltpu.SemaphoreType.DMA,
            pltpu.SemaphoreType.DMA,
            pltpu.SemaphoreType.DMA,
        ],
    )
    def gather(a_hbm, b_hbm, dst2_hbm, src2_hbm, ma_hbm, mb_hbm,
               idxd, idxs, a0, a1, b0, b1,
               gsa0, gsa1, gsb0, gsb1, ssa0, ssa1, ssb0, ssb1):
        wid = lax.axis_index("s") * 2 + lax.axis_index("c")
        cbase = wid * cpt
        pltpu.sync_copy(dst2_hbm.at[pl.ds(cbase, cpt)], idxd)
        pltpu.sync_copy(src2_hbm.at[pl.ds(cbase, cpt)], idxs)

        abufs = (a0, a1)
        bbufs = (b0, b1)
        gsa = (gsa0, gsa1)
        gsb = (gsb0, gsb1)
        ssa = (ssa0, ssa1)
        ssb = (ssb0, ssb1)

        def wait_gather(buf, sem):
            pltpu.make_async_copy(a_hbm.at[pl.ds(0, CHUNK)], buf, sem).wait()

        def start_gather(tbl, t, buf, sem):
            pltpu.async_copy(tbl.at[idxd.at[t] if tbl is a_hbm else idxs.at[t]],
                             buf, sem)

        for par in range(2):
            pltpu.async_copy(a_hbm.at[idxd.at[par]], abufs[par], gsa[par])
            pltpu.async_copy(b_hbm.at[idxs.at[par]], bbufs[par], gsb[par])

        def body(tt, carry):
            sts = []
            for par in range(2):
                t = 2 * tt + par
                rbase = (cbase + t) * CHUNK
                wait_gather(abufs[par], gsa[par])
                sts.append(pltpu.async_copy(abufs[par], ma_hbm.at[pl.ds(rbase, CHUNK)], ssa[par]))
                wait_gather(bbufs[par], gsb[par])
                sts.append(pltpu.async_copy(bbufs[par], mb_hbm.at[pl.ds(rbase, CHUNK)], ssb[par]))
            for par in range(2):
                t = 2 * tt + par
                sts[2 * par].wait()
                pltpu.async_copy(a_hbm.at[idxd.at[t + 2]], abufs[par], gsa[par])
                sts[2 * par + 1].wait()
                pltpu.async_copy(b_hbm.at[idxs.at[t + 2]], bbufs[par], gsb[par])
            return carry

        lax.fori_loop(0, cpt // 2 - 1, body, 0)

        sts = []
        for par in range(2):
            t = cpt - 2 + par
            rbase = (cbase + t) * CHUNK
            wait_gather(abufs[par], gsa[par])
            sts.append(pltpu.async_copy(abufs[par], ma_hbm.at[pl.ds(rbase, CHUNK)], ssa[par]))
            wait_gather(bbufs[par], gsb[par])
            sts.append(pltpu.async_copy(bbufs[par], mb_hbm.at[pl.ds(rbase, CHUNK)], ssb[par]))
        for st in sts:
            st.wait()

    return gather


def _make_scatter(npad, epad):
    SCH = 64
    cps = epad // (16 * SCH)
    rows = npad // 16

    @functools.partial(
        pl.kernel,
        mesh=_SC_MESH,
        compiler_params=_SC_PARAMS,
        out_type=jax.ShapeDtypeStruct((2, npad, H), jnp.float32),
        scratch_types=[
            pltpu.VMEM((1, SCH), jnp.int32),
            pltpu.VMEM((1, SCH), jnp.int32),
            pltpu.VMEM((SCH, H), jnp.float32),
            pltpu.VMEM((SCH, H), jnp.float32),
            pltpu.VMEM_SHARED((npad, H), jnp.float32),
            pltpu.SemaphoreType.DMA,
            pltpu.SemaphoreType.DMA,
            pltpu.SemaphoreType.DMA,
            pltpu.SemaphoreType.DMA,
            pltpu.SemaphoreType.DMA,
            pltpu.SemaphoreType.DMA,
        ],
    )
    def scatter(u2_hbm, dst2_hbm, zeros_hbm, out_hbm,
                i0, i1, u0, u1, acc, li0, li1, lu0, lu1, sc0, sc1):
        c = lax.axis_index("c")
        s = lax.axis_index("s")
        pltpu.sync_copy(zeros_hbm.at[pl.ds(s * rows, rows)],
                        acc.at[pl.ds(s * rows, rows)])
        plsc.subcore_barrier()

        ibufs = (i0, i1)
        ubufs = (u0, u1)
        lisem = (li0, li1)
        lusem = (lu0, lu1)
        ssem = (sc0, sc1)

        def start_loads(t, par):
            pltpu.async_copy(dst2_hbm.at[pl.ds(s * cps + t, 1)], ibufs[par], lisem[par])
            pltpu.async_copy(u2_hbm.at[c, pl.ds((s * cps + t) * SCH, SCH)],
                             ubufs[par], lusem[par])

        def wait_loads(par):
            pltpu.make_async_copy(dst2_hbm.at[pl.ds(0, 1)], ibufs[par], lisem[par]).wait()
            pltpu.make_async_copy(u2_hbm.at[0, pl.ds(0, SCH)], ubufs[par], lusem[par]).wait()

        for par in range(2):
            start_loads(par, par)

        def body(tt, carry):
            scs = []
            for par in range(2):
                wait_loads(par)
                scs.append(pltpu.async_copy(ubufs[par], acc.at[ibufs[par].at[0]],
                                            ssem[par], add=True))
            for par in range(2):
                t = 2 * tt + par
                scs[par].wait()
                start_loads(t + 2, par)
            return carry

        lax.fori_loop(0, cps // 2 - 1, body, 0)

        scs = []
        for par in range(2):
            wait_loads(par)
            scs.append(pltpu.async_copy(ubufs[par], acc.at[ibufs[par].at[0]],
                                        ssem[par], add=True))
        for sc in scs:
            sc.wait()

        plsc.subcore_barrier()
        pltpu.sync_copy(acc.at[pl.ds(s * rows, rows)],
                        out_hbm.at[c, pl.ds(s * rows, rows)])

    return scatter


def _make_deg(npad, epad):
    cps = epad // (16 * CHUNK)
    rows = npad // 16

    @functools.partial(
        pl.kernel,
        mesh=_SC_MESH,
        compiler_params=_SC_PARAMS,
        out_type=jax.ShapeDtypeStruct((2, npad, H), jnp.float32),
        scratch_types=[
            pltpu.VMEM((1, CHUNK), jnp.int32),
            pltpu.VMEM((1, CHUNK), jnp.int32),
            pltpu.VMEM((CHUNK, H), jnp.float32),
            pltpu.VMEM_SHARED((npad, H), jnp.float32),
            pltpu.SemaphoreType.DMA,
            pltpu.SemaphoreType.DMA,
            pltpu.SemaphoreType.DMA,
            pltpu.SemaphoreType.DMA,
        ],
    )
    def deg(ei2_hbm, ones_hbm, zeros_hbm, out_hbm,
            i0, i1, ones_v, acc, li0, li1, sc0, sc1):
        c = lax.axis_index("c")
        s = lax.axis_index("s")
        pltpu.sync_copy(ones_hbm, ones_v)
        pltpu.sync_copy(zeros_hbm.at[pl.ds(s * rows, rows)],
                        acc.at[pl.ds(s * rows, rows)])
        plsc.subcore_barrier()

        ibufs = (i0, i1)
        lisem = (li0, li1)
        ssem = (sc0, sc1)

        def start_load(t, par):
            pltpu.async_copy(ei2_hbm.at[c, pl.ds(s * cps + t, 1)], ibufs[par], lisem[par])

        def wait_load(par):
            pltpu.make_async_copy(ei2_hbm.at[0, pl.ds(0, 1)], ibufs[par], lisem[par]).wait()

        for par in range(2):
            start_load(par, par)

        def body(tt, carry):
            scs = []
            for par in range(2):
                wait_load(par)
                scs.append(pltpu.async_copy(ones_v, acc.at[ibufs[par].at[0]],
                                            ssem[par], add=True))
            for par in range(2):
                t = 2 * tt + par
                scs[par].wait()
                start_load(t + 2, par)
            return carry

        lax.fori_loop(0, cps // 2 - 1, body, 0)

        scs = []
        for par in range(2):
            wait_load(par)
            scs.append(pltpu.async_copy(ones_v, acc.at[ibufs[par].at[0]],
                                        ssem[par], add=True))
        for sc in scs:
            sc.wait()

        plsc.subcore_barrier()
        pltpu.sync_copy(acc.at[pl.ds(s * rows, rows)],
                        out_hbm.at[c, pl.ds(s * rows, rows)])

    return deg


def _conv(left, right, dst, src, dst64, u2_zeros, p, i, deg, gather_fn, scatter_fn):
    econst = p['ee_ln_b'][0] * p['We'][i][0]
    a, b = _pre(right, left, p['Wl'][i], p['bl'][i] + econst, p['Wr'][i])
    ma, mb = gather_fn(a, b, dst, src)
    u2 = _edge(ma, mb, p['lnf_g'][i], p['lnf_b'][i])
    u_seg = scatter_fn(u2, dst64, u2_zeros)
    return _post(u_seg, deg, right, p['Wf'][i], p['bf'][i], p['lnp_g'][i],
                 p['lnp_b'][i], p['Wo1'][i], p['bo1'][i], p['Wo2'][i], p['bo2'][i])


def kernel(constraint_features, edge_indices, edge_features, variable_features, params):
    p = params
    n = variable_features.shape[0]
    e = edge_indices.shape[1]
    npad = ((n + NBLK - 1) // NBLK) * NBLK
    epad = ((e + NW * CHUNK - 1) // (NW * CHUNK)) * (NW * CHUNK)

    cfp = jnp.pad(constraint_features, ((0, npad - n), (0, 0)))
    vfp = jnp.pad(variable_features, ((0, npad - n), (0, 0)))
    eip = jnp.pad(edge_indices, ((0, 0), (0, epad - e)), constant_values=n)
    eip_r = eip.reshape(2, epad // CHUNK, CHUNK)
    eip_r64 = eip.reshape(2, epad // 64, 64)
    dst_c, dst_v = eip_r[0], eip_r[1]
    dst_c64, dst_v64 = eip_r64[0], eip_r64[1]
    acc_zeros = jnp.zeros((npad, H), jnp.float32)
    ones_chunk = jnp.ones((CHUNK, H), jnp.float32)

    gather_fn = _make_gather(npad, epad)
    scatter_fn = _make_scatter(npad, epad)
    deg_fn = _make_deg(npad, epad)

    c = _embed(cfp, p['ce_ln_g'], p['ce_ln_b'], p['ce_w1'], p['ce_b1'],
               p['ce_w2'], p['ce_b2'])
    v = _embed(vfp, p['ve_ln_g'], p['ve_ln_b'], p['ve_w1'], p['ve_b1'],
               p['ve_w2'], p['ve_b2'])

    degs = deg_fn(eip_r, ones_chunk, acc_zeros)
    deg_c, deg_v = degs[0], degs[1]

    for l in range(2):
        c = _conv(v, c, dst_c, dst_v, dst_c64, acc_zeros, p, 2 * l, deg_c,
                  gather_fn, scatter_fn)
        v = _conv(c, v, dst_v, dst_c, dst_v64, acc_zeros, p, 2 * l + 1, deg_v,
                  gather_fn, scatter_fn)

    w2p = jnp.pad(p['out_w2'], ((0, 0), (0, 128 - p['out_w2'].shape[1])))
    out = _head(v, p['out_w1'], p['out_b1'], w2p)
    return out[:n, : p['out_w2'].shape[1]]


# speedup vs baseline: 2.2436x; 2.2436x over previous
import functools

import jax
import jax.numpy as jnp
from jax import lax
from jax.experimental import pallas as pl
from jax.experimental.pallas import tpu as pltpu
from jax.experimental.pallas import tpu_sc as plsc

D = 64
H = 32
NBLK = 1024
EBLK = 2048
CHUNK = 128
NW = 32
EPS = 1e-5

_SC_MESH = plsc.VectorSubcoreMesh(core_axis_name="c", subcore_axis_name="s")
_SC_PARAMS = pltpu.CompilerParams(use_tc_tiling_on_sc=False)


def _embed_body(x_ref, g_ref, b_ref, w1_ref, b1_ref, w2_ref, b2_ref, o_ref):
    x = x_ref[...]
    m = jnp.mean(x, axis=-1, keepdims=True)
    v = jnp.mean((x - m) ** 2, axis=-1, keepdims=True)
    xn = (x - m) * lax.rsqrt(v + EPS) * g_ref[...] + b_ref[...]
    h = jnp.maximum(xn @ w1_ref[...] + b1_ref[...], 0.0)
    o_ref[...] = jnp.maximum(h @ w2_ref[...] + b2_ref[...], 0.0)


def _embed(x, g, b, w1, b1, w2, b2):
    n, f = x.shape
    return pl.pallas_call(
        _embed_body,
        grid=(n // NBLK,),
        in_specs=[
            pl.BlockSpec((NBLK, f), lambda i: (i, 0)),
            pl.BlockSpec((1, f), lambda i: (0, 0)),
            pl.BlockSpec((1, f), lambda i: (0, 0)),
            pl.BlockSpec((f, D), lambda i: (0, 0)),
            pl.BlockSpec((1, D), lambda i: (0, 0)),
            pl.BlockSpec((D, D), lambda i: (0, 0)),
            pl.BlockSpec((1, D), lambda i: (0, 0)),
        ],
        out_specs=pl.BlockSpec((NBLK, D), lambda i: (i, 0)),
        out_shape=jax.ShapeDtypeStruct((n, D), jnp.float32),
    )(x, g[None, :], b[None, :], w1, b1[None, :], w2, b2[None, :])


def _pre_body(r_ref, l_ref, wl_ref, bl_ref, wr_ref, a_ref, b_ref):
    a_ref[...] = r_ref[...] @ wl_ref[...] + bl_ref[...]
    b_ref[...] = l_ref[...] @ wr_ref[...]


def _pre(right, left, wl, bl2, wr):
    n = right.shape[0]
    return pl.pallas_call(
        _pre_body,
        grid=(n // NBLK,),
        in_specs=[
            pl.BlockSpec((NBLK, D), lambda i: (i, 0)),
            pl.BlockSpec((NBLK, D), lambda i: (i, 0)),
            pl.BlockSpec((D, D), lambda i: (0, 0)),
            pl.BlockSpec((1, D), lambda i: (0, 0)),
            pl.BlockSpec((D, D), lambda i: (0, 0)),
        ],
        out_specs=[
            pl.BlockSpec((NBLK, D), lambda i: (i, 0)),
            pl.BlockSpec((NBLK, D), lambda i: (i, 0)),
        ],
        out_shape=[
            jax.ShapeDtypeStruct((n, D), jnp.float32),
            jax.ShapeDtypeStruct((n, D), jnp.float32),
        ],
    )(right, left, wl, bl2[None, :], wr)


def _edge_body(ma_ref, mb_ref, g_ref, b_ref, o_ref):
    m = ma_ref[...] + mb_ref[...]
    r = lax.broadcasted_iota(jnp.int32, (2 * D, 2 * D), 0)
    cc = lax.broadcasted_iota(jnp.int32, (2 * D, 2 * D), 1)
    j2 = jnp.where((r // D) == (cc // D), 1.0 / D, 0.0)
    mu = jax.lax.dot(m, j2, precision=jax.lax.Precision.HIGHEST)
    cen = m - mu
    var = jax.lax.dot(cen * cen, j2, precision=jax.lax.Precision.HIGHEST)
    o_ref[...] = jnp.maximum(
        cen * lax.rsqrt(var + EPS) * g_ref[...] + b_ref[...], 0.0)


def _edge(ma, mb, g, b):
    e = ma.shape[0]
    e2 = e // 2
    ma2 = ma.reshape(e2, 2 * D)
    mb2 = mb.reshape(e2, 2 * D)
    g2 = jnp.concatenate([g, g])[None, :]
    b2 = jnp.concatenate([b, b])[None, :]
    u2 = pl.pallas_call(
        _edge_body,
        grid=(e2 // EBLK,),
        in_specs=[
            pl.BlockSpec((EBLK, 2 * D), lambda i: (i, 0)),
            pl.BlockSpec((EBLK, 2 * D), lambda i: (i, 0)),
            pl.BlockSpec((1, 2 * D), lambda i: (0, 0)),
            pl.BlockSpec((1, 2 * D), lambda i: (0, 0)),
        ],
        out_specs=pl.BlockSpec((EBLK, 2 * D), lambda i: (i, 0)),
        out_shape=jax.ShapeDtypeStruct((e2, 2 * D), jnp.float32),
    )(ma2, mb2, g2, b2)
    return u2.reshape(e, D)


def _post_body(ulo_ref, uhi_ref, deg_ref, r_ref, wf_ref, bf_ref, lg_ref,
               lb_ref, wo1_ref, bo1_ref, wo2_ref, bo2_ref, o_ref):
    u = jnp.concatenate([ulo_ref[0], uhi_ref[0]], axis=-1)
    agg = u @ wf_ref[...] + deg_ref[...][:, :1] * bf_ref[...]
    mu = jnp.mean(agg, axis=-1, keepdims=True)
    var = jnp.mean((agg - mu) ** 2, axis=-1, keepdims=True)
    aggn = (agg - mu) * lax.rsqrt(var + EPS) * lg_ref[...] + lb_ref[...]
    cat = jnp.concatenate([aggn, r_ref[...]], axis=-1)
    h = jnp.maximum(cat @ wo1_ref[...] + bo1_ref[...], 0.0)
    o_ref[...] = h @ wo2_ref[...] + bo2_ref[...]


def _post(u2, deg, right, wf, bf, lg, lb, wo1, bo1, wo2, bo2):
    n = right.shape[0]
    return pl.pallas_call(
        _post_body,
        grid=(n // NBLK,),
        in_specs=[
            pl.BlockSpec((1, NBLK, H), lambda i: (0, i, 0)),
            pl.BlockSpec((1, NBLK, H), lambda i: (1, i, 0)),
            pl.BlockSpec((NBLK, H), lambda i: (i, 0)),
            pl.BlockSpec((NBLK, D), lambda i: (i, 0)),
            pl.BlockSpec((D, D), lambda i: (0, 0)),
            pl.BlockSpec((1, D), lambda i: (0, 0)),
            pl.BlockSpec((1, D), lambda i: (0, 0)),
            pl.BlockSpec((1, D), lambda i: (0, 0)),
            pl.BlockSpec((2 * D, D), lambda i: (0, 0)),
            pl.BlockSpec((1, D), lambda i: (0, 0)),
            pl.BlockSpec((D, D), lambda i: (0, 0)),
            pl.BlockSpec((1, D), lambda i: (0, 0)),
        ],
        out_specs=pl.BlockSpec((NBLK, D), lambda i: (i, 0)),
        out_shape=jax.ShapeDtypeStruct((n, D), jnp.float32),
    )(u2, u2, deg, right, wf, bf[None, :], lg[None, :], lb[None, :],
      wo1, bo1[None, :], wo2, bo2[None, :])


def _head_body(x_ref, w1_ref, b1_ref, w2_ref, o_ref):
    h = jnp.maximum(x_ref[...] @ w1_ref[...] + b1_ref[...], 0.0)
    o_ref[...] = h @ w2_ref[...]


def _head(x, w1, b1, w2p):
    n = x.shape[0]
    return pl.pallas_call(
        _head_body,
        grid=(n // NBLK,),
        in_specs=[
            pl.BlockSpec((NBLK, D), lambda i: (i, 0)),
            pl.BlockSpec((D, D), lambda i: (0, 0)),
            pl.BlockSpec((1, D), lambda i: (0, 0)),
            pl.BlockSpec((D, 128), lambda i: (0, 0)),
        ],
        out_specs=pl.BlockSpec((NBLK, 128), lambda i: (i, 0)),
        out_shape=jax.ShapeDtypeStruct((n, 128), jnp.float32),
    )(x, w1, b1[None, :], w2p)


def _make_gather(npad, epad):
    cpt = epad // (NW * CHUNK)

    @functools.partial(
        pl.kernel,
        mesh=_SC_MESH,
        compiler_params=_SC_PARAMS,
        out_type=[
            jax.ShapeDtypeStruct((epad, D), jnp.float32),
            jax.ShapeDtypeStruct((epad, D), jnp.float32),
        ],
        scratch_types=[
            pltpu.VMEM((cpt, CHUNK), jnp.int32),
            pltpu.VMEM((cpt, CHUNK), jnp.int32),
            pltpu.VMEM((CHUNK, D), jnp.float32),
            pltpu.VMEM((CHUNK, D), jnp.float32),
            pltpu.VMEM((CHUNK, D), jnp.float32),
            pltpu.VMEM((CHUNK, D), jnp.float32),
            pltpu.SemaphoreType.DMA,
            pltpu.SemaphoreType.DMA,
            pltpu.SemaphoreType.DMA,
            pltpu.SemaphoreType.DMA,
            pltpu.SemaphoreType.DMA,
            pltpu.SemaphoreType.DMA,
            pltpu.SemaphoreType.DMA,
            pltpu.SemaphoreType.DMA,
        ],
    )
    def gather(a_hbm, b_hbm, dst2_hbm, src2_hbm, ma_hbm, mb_hbm,
               idxd, idxs, a0, a1, b0, b1,
               gsa0, gsa1, gsb0, gsb1, ssa0, ssa1, ssb0, ssb1):
        wid = lax.axis_index("s") * 2 + lax.axis_index("c")
        cbase = wid * cpt
        pltpu.sync_copy(dst2_hbm.at[pl.ds(cbase, cpt)], idxd)
        pltpu.sync_copy(src2_hbm.at[pl.ds(cbase, cpt)], idxs)

        abufs = (a0, a1)
        bbufs = (b0, b1)
        gsa = (gsa0, gsa1)
        gsb = (gsb0, gsb1)
        ssa = (ssa0, ssa1)
        ssb = (ssb0, ssb1)

        def wait_gather(buf, sem):
            pltpu.make_async_copy(a_hbm.at[pl.ds(0, CHUNK)], buf, sem).wait()

        def start_gather(tbl, t, buf, sem):
            pltpu.async_copy(tbl.at[idxd.at[t] if tbl is a_hbm else idxs.at[t]],
                             buf, sem)

        for par in range(2):
            pltpu.async_copy(a_hbm.at[idxd.at[par]], abufs[par], gsa[par])
            pltpu.async_copy(b_hbm.at[idxs.at[par]], bbufs[par], gsb[par])

        def body(tt, carry):
            sts = []
            for par in range(2):
                t = 2 * tt + par
                rbase = (cbase + t) * CHUNK
                wait_gather(abufs[par], gsa[par])
                sts.append(pltpu.async_copy(abufs[par], ma_hbm.at[pl.ds(rbase, CHUNK)], ssa[par]))
                wait_gather(bbufs[par], gsb[par])
                sts.append(pltpu.async_copy(bbufs[par], mb_hbm.at[pl.ds(rbase, CHUNK)], ssb[par]))
            for par in range(2):
                t = 2 * tt + par
                sts[2 * par].wait()
                pltpu.async_copy(a_hbm.at[idxd.at[t + 2]], abufs[par], gsa[par])
                sts[2 * par + 1].wait()
                pltpu.async_copy(b_hbm.at[idxs.at[t + 2]], bbufs[par], gsb[par])
            return carry

        lax.fori_loop(0, cpt // 2 - 1, body, 0)

        sts = []
        for par in range(2):
            t = cpt - 2 + par
            rbase = (cbase + t) * CHUNK
            wait_gather(abufs[par], gsa[par])
            sts.append(pltpu.async_copy(abufs[par], ma_hbm.at[pl.ds(rbase, CHUNK)], ssa[par]))
            wait_gather(bbufs[par], gsb[par])
            sts.append(pltpu.async_copy(bbufs[par], mb_hbm.at[pl.ds(rbase, CHUNK)], ssb[par]))
        for st in sts:
            st.wait()

    return gather


def _make_scatter(npad, epad):
    SCH = 64
    cps = epad // (16 * SCH)
    rows = npad // 16

    @functools.partial(
        pl.kernel,
        mesh=_SC_MESH,
        compiler_params=_SC_PARAMS,
        out_type=jax.ShapeDtypeStruct((2, npad, H), jnp.float32),
        scratch_types=[
            pltpu.VMEM((1, SCH), jnp.int32),
            pltpu.VMEM((1, SCH), jnp.int32),
            pltpu.VMEM((SCH, H), jnp.float32),
            pltpu.VMEM((SCH, H), jnp.float32),
            pltpu.VMEM_SHARED((npad, H), jnp.float32),
            pltpu.SemaphoreType.DMA,
            pltpu.SemaphoreType.DMA,
            pltpu.SemaphoreType.DMA,
            pltpu.SemaphoreType.DMA,
            pltpu.SemaphoreType.DMA,
            pltpu.SemaphoreType.DMA,
        ],
    )
    def scatter(u2_hbm, dst2_hbm, zeros_hbm, out_hbm,
                i0, i1, u0, u1, acc, li0, li1, lu0, lu1, sc0, sc1):
        c = lax.axis_index("c")
        s = lax.axis_index("s")
        pltpu.sync_copy(zeros_hbm.at[pl.ds(s * rows, rows)],
                        acc.at[pl.ds(s * rows, rows)])
        plsc.subcore_barrier()

        ibufs = (i0, i1)
        ubufs = (u0, u1)
        lisem = (li0, li1)
        lusem = (lu0, lu1)
        ssem = (sc0, sc1)

        def start_loads(t, par):
            pltpu.async_copy(dst2_hbm.at[pl.ds(s * cps + t, 1)], ibufs[par], lisem[par])
            pltpu.async_copy(u2_hbm.at[pl.ds((s * cps + t) * SCH, SCH), pl.ds(c * H, H)],
                             ubufs[par], lusem[par])

        def wait_loads(par):
            pltpu.make_async_copy(dst2_hbm.at[pl.ds(0, 1)], ibufs[par], lisem[par]).wait()
            pltpu.make_async_copy(u2_hbm.at[pl.ds(0, SCH), pl.ds(0, H)], ubufs[par], lusem[par]).wait()

        for par in range(2):
            start_loads(par, par)

        def body(tt, carry):
            scs = []
            for par in range(2):
                wait_loads(par)
                scs.append(pltpu.async_copy(ubufs[par], acc.at[ibufs[par].at[0]],
                                            ssem[par], add=True))
            for par in range(2):
                t = 2 * tt + par
                scs[par].wait()
                start_loads(t + 2, par)
            return carry

        lax.fori_loop(0, cps // 2 - 1, body, 0)

        scs = []
        for par in range(2):
            wait_loads(par)
            scs.append(pltpu.async_copy(ubufs[par], acc.at[ibufs[par].at[0]],
                                        ssem[par], add=True))
        for sc in scs:
            sc.wait()

        plsc.subcore_barrier()
        pltpu.sync_copy(acc.at[pl.ds(s * rows, rows)],
                        out_hbm.at[c, pl.ds(s * rows, rows)])

    return scatter


def _make_deg(npad, epad):
    cps = epad // (16 * CHUNK)
    rows = npad // 16

    @functools.partial(
        pl.kernel,
        mesh=_SC_MESH,
        compiler_params=_SC_PARAMS,
        out_type=jax.ShapeDtypeStruct((2, npad, H), jnp.float32),
        scratch_types=[
            pltpu.VMEM((1, CHUNK), jnp.int32),
            pltpu.VMEM((1, CHUNK), jnp.int32),
            pltpu.VMEM((CHUNK, H), jnp.float32),
            pltpu.VMEM_SHARED((npad, H), jnp.float32),
            pltpu.SemaphoreType.DMA,
            pltpu.SemaphoreType.DMA,
            pltpu.SemaphoreType.DMA,
            pltpu.SemaphoreType.DMA,
        ],
    )
    def deg(ei2_hbm, ones_hbm, zeros_hbm, out_hbm,
            i0, i1, ones_v, acc, li0, li1, sc0, sc1):
        c = lax.axis_index("c")
        s = lax.axis_index("s")
        pltpu.sync_copy(ones_hbm, ones_v)
        pltpu.sync_copy(zeros_hbm.at[pl.ds(s * rows, rows)],
                        acc.at[pl.ds(s * rows, rows)])
        plsc.subcore_barrier()

        ibufs = (i0, i1)
        lisem = (li0, li1)
        ssem = (sc0, sc1)

        def start_load(t, par):
            pltpu.async_copy(ei2_hbm.at[c, pl.ds(s * cps + t, 1)], ibufs[par], lisem[par])

        def wait_load(par):
            pltpu.make_async_copy(ei2_hbm.at[0, pl.ds(0, 1)], ibufs[par], lisem[par]).wait()

        for par in range(2):
            start_load(par, par)

        def body(tt, carry):
            scs = []
            for par in range(2):
                wait_load(par)
                scs.append(pltpu.async_copy(ones_v, acc.at[ibufs[par].at[0]],
                                            ssem[par], add=True))
            for par in range(2):
                t = 2 * tt + par
                scs[par].wait()
                start_load(t + 2, par)
            return carry

        lax.fori_loop(0, cps // 2 - 1, body, 0)

        scs = []
        for par in range(2):
            wait_load(par)
            scs.append(pltpu.async_copy(ones_v, acc.at[ibufs[par].at[0]],
                                        ssem[par], add=True))
        for sc in scs:
            sc.wait()

        plsc.subcore_barrier()
        pltpu.sync_copy(acc.at[pl.ds(s * rows, rows)],
                        out_hbm.at[c, pl.ds(s * rows, rows)])

    return deg


def _conv(left, right, dst, src, dst64, u2_zeros, p, i, deg, gather_fn, scatter_fn):
    econst = p['ee_ln_b'][0] * p['We'][i][0]
    a, b = _pre(right, left, p['Wl'][i], p['bl'][i] + econst, p['Wr'][i])
    ma, mb = gather_fn(a, b, dst, src)
    u2 = _edge(ma, mb, p['lnf_g'][i], p['lnf_b'][i])
    u_seg = scatter_fn(u2, dst64, u2_zeros)
    return _post(u_seg, deg, right, p['Wf'][i], p['bf'][i], p['lnp_g'][i],
                 p['lnp_b'][i], p['Wo1'][i], p['bo1'][i], p['Wo2'][i], p['bo2'][i])


def kernel(constraint_features, edge_indices, edge_features, variable_features, params):
    p = params
    n = variable_features.shape[0]
    e = edge_indices.shape[1]
    npad = ((n + NBLK - 1) // NBLK) * NBLK
    epad = ((e + NW * CHUNK - 1) // (NW * CHUNK)) * (NW * CHUNK)

    cfp = jnp.pad(constraint_features, ((0, npad - n), (0, 0)))
    vfp = jnp.pad(variable_features, ((0, npad - n), (0, 0)))
    eip = jnp.pad(edge_indices, ((0, 0), (0, epad - e)), constant_values=n)
    eip_r = eip.reshape(2, epad // CHUNK, CHUNK)
    eip_r64 = eip.reshape(2, epad // 64, 64)
    dst_c, dst_v = eip_r[0], eip_r[1]
    dst_c64, dst_v64 = eip_r64[0], eip_r64[1]
    acc_zeros = jnp.zeros((npad, H), jnp.float32)
    ones_chunk = jnp.ones((CHUNK, H), jnp.float32)

    gather_fn = _make_gather(npad, epad)
    scatter_fn = _make_scatter(npad, epad)
    deg_fn = _make_deg(npad, epad)

    c = _embed(cfp, p['ce_ln_g'], p['ce_ln_b'], p['ce_w1'], p['ce_b1'],
               p['ce_w2'], p['ce_b2'])
    v = _embed(vfp, p['ve_ln_g'], p['ve_ln_b'], p['ve_w1'], p['ve_b1'],
               p['ve_w2'], p['ve_b2'])

    degs = deg_fn(eip_r, ones_chunk, acc_zeros)
    deg_c, deg_v = degs[0], degs[1]

    for l in range(2):
        c = _conv(v, c, dst_c, dst_v, dst_c64, acc_zeros, p, 2 * l, deg_c,
                  gather_fn, scatter_fn)
        v = _conv(c, v, dst_v, dst_c, dst_v64, acc_zeros, p, 2 * l + 1, deg_v,
                  gather_fn, scatter_fn)

    w2p = jnp.pad(p['out_w2'], ((0, 0), (0, 128 - p['out_w2'].shape[1])))
    out = _head(v, p['out_w1'], p['out_b1'], w2p)
    return out[:n, : p['out_w2'].shape[1]]


# speedup vs baseline: 2.3784x; 1.0601x over previous
import functools

import jax
import jax.numpy as jnp
from jax import lax
from jax.experimental import pallas as pl
from jax.experimental.pallas import tpu as pltpu
from jax.experimental.pallas import tpu_sc as plsc

D = 64
H = 32
NBLK = 1024
EBLK = 4096
CHUNK = 128
NW = 32
EPS = 1e-5

_SC_MESH = plsc.VectorSubcoreMesh(core_axis_name="c", subcore_axis_name="s")
_SC_PARAMS = pltpu.CompilerParams(use_tc_tiling_on_sc=False)


def _embed_body(x_ref, g_ref, b_ref, w1_ref, b1_ref, w2_ref, b2_ref, o_ref):
    x = x_ref[...]
    m = jnp.mean(x, axis=-1, keepdims=True)
    v = jnp.mean((x - m) ** 2, axis=-1, keepdims=True)
    xn = (x - m) * lax.rsqrt(v + EPS) * g_ref[...] + b_ref[...]
    h = jnp.maximum(xn @ w1_ref[...] + b1_ref[...], 0.0)
    o_ref[...] = jnp.maximum(h @ w2_ref[...] + b2_ref[...], 0.0)


def _embed(x, g, b, w1, b1, w2, b2):
    n, f = x.shape
    return pl.pallas_call(
        _embed_body,
        grid=(n // NBLK,),
        in_specs=[
            pl.BlockSpec((NBLK, f), lambda i: (i, 0)),
            pl.BlockSpec((1, f), lambda i: (0, 0)),
            pl.BlockSpec((1, f), lambda i: (0, 0)),
            pl.BlockSpec((f, D), lambda i: (0, 0)),
            pl.BlockSpec((1, D), lambda i: (0, 0)),
            pl.BlockSpec((D, D), lambda i: (0, 0)),
            pl.BlockSpec((1, D), lambda i: (0, 0)),
        ],
        out_specs=pl.BlockSpec((NBLK, D), lambda i: (i, 0)),
        out_shape=jax.ShapeDtypeStruct((n, D), jnp.float32),
    )(x, g[None, :], b[None, :], w1, b1[None, :], w2, b2[None, :])


def _pre_body(r_ref, l_ref, wl_ref, bl_ref, wr_ref, a_ref, b_ref):
    a_ref[...] = r_ref[...] @ wl_ref[...] + bl_ref[...]
    b_ref[...] = l_ref[...] @ wr_ref[...]


def _pre(right, left, wl, bl2, wr):
    n = right.shape[0]
    return pl.pallas_call(
        _pre_body,
        grid=(n // NBLK,),
        in_specs=[
            pl.BlockSpec((NBLK, D), lambda i: (i, 0)),
            pl.BlockSpec((NBLK, D), lambda i: (i, 0)),
            pl.BlockSpec((D, D), lambda i: (0, 0)),
            pl.BlockSpec((1, D), lambda i: (0, 0)),
            pl.BlockSpec((D, D), lambda i: (0, 0)),
        ],
        out_specs=[
            pl.BlockSpec((NBLK, D), lambda i: (i, 0)),
            pl.BlockSpec((NBLK, D), lambda i: (i, 0)),
        ],
        out_shape=[
            jax.ShapeDtypeStruct((n, D), jnp.float32),
            jax.ShapeDtypeStruct((n, D), jnp.float32),
        ],
    )(right, left, wl, bl2[None, :], wr)


def _edge_body(ma_ref, mb_ref, g_ref, b_ref, o_ref):
    m = ma_ref[...] + mb_ref[...]
    r = lax.broadcasted_iota(jnp.int32, (2 * D, 2 * D), 0)
    cc = lax.broadcasted_iota(jnp.int32, (2 * D, 2 * D), 1)
    j2 = jnp.where((r // D) == (cc // D), 1.0 / D, 0.0)
    mu = jax.lax.dot(m, j2, precision=jax.lax.Precision.HIGHEST)
    cen = m - mu
    var = jax.lax.dot(cen * cen, j2, precision=jax.lax.Precision.HIGHEST)
    o_ref[...] = jnp.maximum(
        cen * lax.rsqrt(var + EPS) * g_ref[...] + b_ref[...], 0.0)


def _edge(ma, mb, g, b):
    e = ma.shape[0]
    e2 = e // 2
    ma2 = ma.reshape(e2, 2 * D)
    mb2 = mb.reshape(e2, 2 * D)
    g2 = jnp.concatenate([g, g])[None, :]
    b2 = jnp.concatenate([b, b])[None, :]
    u2 = pl.pallas_call(
        _edge_body,
        grid=(e2 // EBLK,),
        in_specs=[
            pl.BlockSpec((EBLK, 2 * D), lambda i: (i, 0)),
            pl.BlockSpec((EBLK, 2 * D), lambda i: (i, 0)),
            pl.BlockSpec((1, 2 * D), lambda i: (0, 0)),
            pl.BlockSpec((1, 2 * D), lambda i: (0, 0)),
        ],
        out_specs=pl.BlockSpec((EBLK, 2 * D), lambda i: (i, 0)),
        out_shape=jax.ShapeDtypeStruct((e2, 2 * D), jnp.float32),
    )(ma2, mb2, g2, b2)
    return u2.reshape(e, D)


def _post_body(ulo_ref, uhi_ref, deg_ref, r_ref, wf_ref, bf_ref, lg_ref,
               lb_ref, wo1_ref, bo1_ref, wo2_ref, bo2_ref, o_ref):
    u = jnp.concatenate([ulo_ref[0], uhi_ref[0]], axis=-1)
    agg = u @ wf_ref[...] + deg_ref[...][:, :1] * bf_ref[...]
    mu = jnp.mean(agg, axis=-1, keepdims=True)
    var = jnp.mean((agg - mu) ** 2, axis=-1, keepdims=True)
    aggn = (agg - mu) * lax.rsqrt(var + EPS) * lg_ref[...] + lb_ref[...]
    cat = jnp.concatenate([aggn, r_ref[...]], axis=-1)
    h = jnp.maximum(cat @ wo1_ref[...] + bo1_ref[...], 0.0)
    o_ref[...] = h @ wo2_ref[...] + bo2_ref[...]


def _post(u2, deg, right, wf, bf, lg, lb, wo1, bo1, wo2, bo2):
    n = right.shape[0]
    return pl.pallas_call(
        _post_body,
        grid=(n // NBLK,),
        in_specs=[
            pl.BlockSpec((1, NBLK, H), lambda i: (0, i, 0)),
            pl.BlockSpec((1, NBLK, H), lambda i: (1, i, 0)),
            pl.BlockSpec((NBLK, H), lambda i: (i, 0)),
            pl.BlockSpec((NBLK, D), lambda i: (i, 0)),
            pl.BlockSpec((D, D), lambda i: (0, 0)),
            pl.BlockSpec((1, D), lambda i: (0, 0)),
            pl.BlockSpec((1, D), lambda i: (0, 0)),
            pl.BlockSpec((1, D), lambda i: (0, 0)),
            pl.BlockSpec((2 * D, D), lambda i: (0, 0)),
            pl.BlockSpec((1, D), lambda i: (0, 0)),
            pl.BlockSpec((D, D), lambda i: (0, 0)),
            pl.BlockSpec((1, D), lambda i: (0, 0)),
        ],
        out_specs=pl.BlockSpec((NBLK, D), lambda i: (i, 0)),
        out_shape=jax.ShapeDtypeStruct((n, D), jnp.float32),
    )(u2, u2, deg, right, wf, bf[None, :], lg[None, :], lb[None, :],
      wo1, bo1[None, :], wo2, bo2[None, :])


def _head_body(x_ref, w1_ref, b1_ref, w2_ref, o_ref):
    h = jnp.maximum(x_ref[...] @ w1_ref[...] + b1_ref[...], 0.0)
    o_ref[...] = h @ w2_ref[...]


def _head(x, w1, b1, w2p):
    n = x.shape[0]
    return pl.pallas_call(
        _head_body,
        grid=(n // NBLK,),
        in_specs=[
            pl.BlockSpec((NBLK, D), lambda i: (i, 0)),
            pl.BlockSpec((D, D), lambda i: (0, 0)),
            pl.BlockSpec((1, D), lambda i: (0, 0)),
            pl.BlockSpec((D, 128), lambda i: (0, 0)),
        ],
        out_specs=pl.BlockSpec((NBLK, 128), lambda i: (i, 0)),
        out_shape=jax.ShapeDtypeStruct((n, 128), jnp.float32),
    )(x, w1, b1[None, :], w2p)


def _make_gather(npad, epad):
    cpt = epad // (NW * CHUNK)

    @functools.partial(
        pl.kernel,
        mesh=_SC_MESH,
        compiler_params=_SC_PARAMS,
        out_type=[
            jax.ShapeDtypeStruct((epad, D), jnp.float32),
            jax.ShapeDtypeStruct((epad, D), jnp.float32),
        ],
        scratch_types=[
            pltpu.VMEM((cpt, CHUNK), jnp.int32),
            pltpu.VMEM((cpt, CHUNK), jnp.int32),
            pltpu.VMEM((CHUNK, D), jnp.float32),
            pltpu.VMEM((CHUNK, D), jnp.float32),
            pltpu.VMEM((CHUNK, D), jnp.float32),
            pltpu.VMEM((CHUNK, D), jnp.float32),
            pltpu.SemaphoreType.DMA,
            pltpu.SemaphoreType.DMA,
            pltpu.SemaphoreType.DMA,
            pltpu.SemaphoreType.DMA,
            pltpu.SemaphoreType.DMA,
            pltpu.SemaphoreType.DMA,
            pltpu.SemaphoreType.DMA,
            pltpu.SemaphoreType.DMA,
        ],
    )
    def gather(a_hbm, b_hbm, dst2_hbm, src2_hbm, ma_hbm, mb_hbm,
               idxd, idxs, a0, a1, b0, b1,
               gsa0, gsa1, gsb0, gsb1, ssa0, ssa1, ssb0, ssb1):
        wid = lax.axis_index("s") * 2 + lax.axis_index("c")
        cbase = wid * cpt
        pltpu.sync_copy(dst2_hbm.at[pl.ds(cbase, cpt)], idxd)
        pltpu.sync_copy(src2_hbm.at[pl.ds(cbase, cpt)], idxs)

        abufs = (a0, a1)
        bbufs = (b0, b1)
        gsa = (gsa0, gsa1)
        gsb = (gsb0, gsb1)
        ssa = (ssa0, ssa1)
        ssb = (ssb0, ssb1)

        def wait_gather(buf, sem):
            pltpu.make_async_copy(a_hbm.at[pl.ds(0, CHUNK)], buf, sem).wait()

        def start_gather(tbl, t, buf, sem):
            pltpu.async_copy(tbl.at[idxd.at[t] if tbl is a_hbm else idxs.at[t]],
                             buf, sem)

        for par in range(2):
            pltpu.async_copy(a_hbm.at[idxd.at[par]], abufs[par], gsa[par])
            pltpu.async_copy(b_hbm.at[idxs.at[par]], bbufs[par], gsb[par])

        def body(tt, carry):
            sts = []
            for par in range(2):
                t = 2 * tt + par
                rbase = (cbase + t) * CHUNK
                wait_gather(abufs[par], gsa[par])
                sts.append(pltpu.async_copy(abufs[par], ma_hbm.at[pl.ds(rbase, CHUNK)], ssa[par]))
                wait_gather(bbufs[par], gsb[par])
                sts.append(pltpu.async_copy(bbufs[par], mb_hbm.at[pl.ds(rbase, CHUNK)], ssb[par]))
            for par in range(2):
                t = 2 * tt + par
                sts[2 * par].wait()
                pltpu.async_copy(a_hbm.at[idxd.at[t + 2]], abufs[par], gsa[par])
                sts[2 * par + 1].wait()
                pltpu.async_copy(b_hbm.at[idxs.at[t + 2]], bbufs[par], gsb[par])
            return carry

        lax.fori_loop(0, cpt // 2 - 1, body, 0)

        sts = []
        for par in range(2):
            t = cpt - 2 + par
            rbase = (cbase + t) * CHUNK
            wait_gather(abufs[par], gsa[par])
            sts.append(pltpu.async_copy(abufs[par], ma_hbm.at[pl.ds(rbase, CHUNK)], ssa[par]))
            wait_gather(bbufs[par], gsb[par])
            sts.append(pltpu.async_copy(bbufs[par], mb_hbm.at[pl.ds(rbase, CHUNK)], ssb[par]))
        for st in sts:
            st.wait()

    return gather


def _make_scatter(npad, epad):
    SCH = 128
    cps = epad // (16 * SCH)
    rows = npad // 16

    @functools.partial(
        pl.kernel,
        mesh=_SC_MESH,
        compiler_params=_SC_PARAMS,
        out_type=jax.ShapeDtypeStruct((2, npad, H), jnp.float32),
        scratch_types=[
            pltpu.VMEM((1, SCH), jnp.int32),
            pltpu.VMEM((1, SCH), jnp.int32),
            pltpu.VMEM((SCH, H), jnp.float32),
            pltpu.VMEM((SCH, H), jnp.float32),
            pltpu.VMEM_SHARED((npad, H), jnp.float32),
            pltpu.SemaphoreType.DMA,
            pltpu.SemaphoreType.DMA,
            pltpu.SemaphoreType.DMA,
            pltpu.SemaphoreType.DMA,
            pltpu.SemaphoreType.DMA,
            pltpu.SemaphoreType.DMA,
        ],
    )
    def scatter(u2_hbm, dst2_hbm, zeros_hbm, out_hbm,
                i0, i1, u0, u1, acc, li0, li1, lu0, lu1, sc0, sc1):
        c = lax.axis_index("c")
        s = lax.axis_index("s")
        pltpu.sync_copy(zeros_hbm.at[pl.ds(s * rows, rows)],
                        acc.at[pl.ds(s * rows, rows)])
        plsc.subcore_barrier()

        ibufs = (i0, i1)
        ubufs = (u0, u1)
        lisem = (li0, li1)
        lusem = (lu0, lu1)
        ssem = (sc0, sc1)

        def start_loads(t, par):
            pltpu.async_copy(dst2_hbm.at[pl.ds(s * cps + t, 1)], ibufs[par], lisem[par])
            pltpu.async_copy(u2_hbm.at[pl.ds((s * cps + t) * SCH, SCH), pl.ds(c * H, H)],
                             ubufs[par], lusem[par])

        def wait_loads(par):
            pltpu.make_async_copy(dst2_hbm.at[pl.ds(0, 1)], ibufs[par], lisem[par]).wait()
            pltpu.make_async_copy(u2_hbm.at[pl.ds(0, SCH), pl.ds(0, H)], ubufs[par], lusem[par]).wait()

        for par in range(2):
            start_loads(par, par)

        def body(tt, carry):
            scs = []
            for par in range(2):
                wait_loads(par)
                scs.append(pltpu.async_copy(ubufs[par], acc.at[ibufs[par].at[0]],
                                            ssem[par], add=True))
            for par in range(2):
                t = 2 * tt + par
                scs[par].wait()
                start_loads(t + 2, par)
            return carry

        lax.fori_loop(0, cps // 2 - 1, body, 0)

        scs = []
        for par in range(2):
            wait_loads(par)
            scs.append(pltpu.async_copy(ubufs[par], acc.at[ibufs[par].at[0]],
                                        ssem[par], add=True))
        for sc in scs:
            sc.wait()

        plsc.subcore_barrier()
        pltpu.sync_copy(acc.at[pl.ds(s * rows, rows)],
                        out_hbm.at[c, pl.ds(s * rows, rows)])

    return scatter


def _make_deg(npad, epad):
    cps = epad // (16 * CHUNK)
    rows = npad // 16

    @functools.partial(
        pl.kernel,
        mesh=_SC_MESH,
        compiler_params=_SC_PARAMS,
        out_type=jax.ShapeDtypeStruct((2, npad, H), jnp.float32),
        scratch_types=[
            pltpu.VMEM((1, CHUNK), jnp.int32),
            pltpu.VMEM((1, CHUNK), jnp.int32),
            pltpu.VMEM((CHUNK, H), jnp.float32),
            pltpu.VMEM_SHARED((npad, H), jnp.float32),
            pltpu.SemaphoreType.DMA,
            pltpu.SemaphoreType.DMA,
            pltpu.SemaphoreType.DMA,
            pltpu.SemaphoreType.DMA,
        ],
    )
    def deg(ei2_hbm, ones_hbm, zeros_hbm, out_hbm,
            i0, i1, ones_v, acc, li0, li1, sc0, sc1):
        c = lax.axis_index("c")
        s = lax.axis_index("s")
        pltpu.sync_copy(ones_hbm, ones_v)
        pltpu.sync_copy(zeros_hbm.at[pl.ds(s * rows, rows)],
                        acc.at[pl.ds(s * rows, rows)])
        plsc.subcore_barrier()

        ibufs = (i0, i1)
        lisem = (li0, li1)
        ssem = (sc0, sc1)

        def start_load(t, par):
            pltpu.async_copy(ei2_hbm.at[c, pl.ds(s * cps + t, 1)], ibufs[par], lisem[par])

        def wait_load(par):
            pltpu.make_async_copy(ei2_hbm.at[0, pl.ds(0, 1)], ibufs[par], lisem[par]).wait()

        for par in range(2):
            start_load(par, par)

        def body(tt, carry):
            scs = []
            for par in range(2):
                wait_load(par)
                scs.append(pltpu.async_copy(ones_v, acc.at[ibufs[par].at[0]],
                                            ssem[par], add=True))
            for par in range(2):
                t = 2 * tt + par
                scs[par].wait()
                start_load(t + 2, par)
            return carry

        lax.fori_loop(0, cps // 2 - 1, body, 0)

        scs = []
        for par in range(2):
            wait_load(par)
            scs.append(pltpu.async_copy(ones_v, acc.at[ibufs[par].at[0]],
                                        ssem[par], add=True))
        for sc in scs:
            sc.wait()

        plsc.subcore_barrier()
        pltpu.sync_copy(acc.at[pl.ds(s * rows, rows)],
                        out_hbm.at[c, pl.ds(s * rows, rows)])

    return deg


def _conv(left, right, dst, src, dst64, u2_zeros, p, i, deg, gather_fn, scatter_fn):
    econst = p['ee_ln_b'][0] * p['We'][i][0]
    a, b = _pre(right, left, p['Wl'][i], p['bl'][i] + econst, p['Wr'][i])
    ma, mb = gather_fn(a, b, dst, src)
    u2 = _edge(ma, mb, p['lnf_g'][i], p['lnf_b'][i])
    u_seg = scatter_fn(u2, dst, u2_zeros)
    return _post(u_seg, deg, right, p['Wf'][i], p['bf'][i], p['lnp_g'][i],
                 p['lnp_b'][i], p['Wo1'][i], p['bo1'][i], p['Wo2'][i], p['bo2'][i])


def kernel(constraint_features, edge_indices, edge_features, variable_features, params):
    p = params
    n = variable_features.shape[0]
    e = edge_indices.shape[1]
    npad = ((n + NBLK - 1) // NBLK) * NBLK
    epad = ((e + NW * CHUNK - 1) // (NW * CHUNK)) * (NW * CHUNK)

    cfp = jnp.pad(constraint_features, ((0, npad - n), (0, 0)))
    vfp = jnp.pad(variable_features, ((0, npad - n), (0, 0)))
    eip = jnp.pad(edge_indices, ((0, 0), (0, epad - e)), constant_values=n)
    eip_r = eip.reshape(2, epad // CHUNK, CHUNK)
    eip_r64 = eip.reshape(2, epad // 64, 64)
    dst_c, dst_v = eip_r[0], eip_r[1]
    dst_c64, dst_v64 = eip_r64[0], eip_r64[1]
    acc_zeros = jnp.zeros((npad, H), jnp.float32)
    ones_chunk = jnp.ones((CHUNK, H), jnp.float32)

    gather_fn = _make_gather(npad, epad)
    scatter_fn = _make_scatter(npad, epad)
    deg_fn = _make_deg(npad, epad)

    c = _embed(cfp, p['ce_ln_g'], p['ce_ln_b'], p['ce_w1'], p['ce_b1'],
               p['ce_w2'], p['ce_b2'])
    v = _embed(vfp, p['ve_ln_g'], p['ve_ln_b'], p['ve_w1'], p['ve_b1'],
               p['ve_w2'], p['ve_b2'])

    degs = deg_fn(eip_r, ones_chunk, acc_zeros)
    deg_c, deg_v = degs[0], degs[1]

    for l in range(2):
        c = _conv(v, c, dst_c, dst_v, dst_c64, acc_zeros, p, 2 * l, deg_c,
                  gather_fn, scatter_fn)
        v = _conv(c, v, dst_v, dst_c, dst_v64, acc_zeros, p, 2 * l + 1, deg_v,
                  gather_fn, scatter_fn)

    w2p = jnp.pad(p['out_w2'], ((0, 0), (0, 128 - p['out_w2'].shape[1])))
    out = _head(v, p['out_w1'], p['out_b1'], w2p)
    return out[:n, : p['out_w2'].shape[1]]


# speedup vs baseline: 3.0967x; 1.3020x over previous
import functools

import jax
import jax.numpy as jnp
from jax import lax
from jax.experimental import pallas as pl
from jax.experimental.pallas import tpu as pltpu
from jax.experimental.pallas import tpu_sc as plsc

D = 64
H = 32
NBLK = 1024
EBLK = 4096
CHUNK = 128
NW = 32
EPS = 1e-5

_SC_MESH = plsc.VectorSubcoreMesh(core_axis_name="c", subcore_axis_name="s")
_SC_PARAMS = pltpu.CompilerParams(use_tc_tiling_on_sc=False)


def _embed_body(x_ref, g_ref, b_ref, w1_ref, b1_ref, w2_ref, b2_ref, o_ref):
    x = x_ref[...]
    m = jnp.mean(x, axis=-1, keepdims=True)
    v = jnp.mean((x - m) ** 2, axis=-1, keepdims=True)
    xn = (x - m) * lax.rsqrt(v + EPS) * g_ref[...] + b_ref[...]
    h = jnp.maximum(xn @ w1_ref[...] + b1_ref[...], 0.0)
    o_ref[...] = jnp.maximum(h @ w2_ref[...] + b2_ref[...], 0.0)


def _embed(x, g, b, w1, b1, w2, b2):
    n, f = x.shape
    return pl.pallas_call(
        _embed_body,
        grid=(n // NBLK,),
        in_specs=[
            pl.BlockSpec((NBLK, f), lambda i: (i, 0)),
            pl.BlockSpec((1, f), lambda i: (0, 0)),
            pl.BlockSpec((1, f), lambda i: (0, 0)),
            pl.BlockSpec((f, D), lambda i: (0, 0)),
            pl.BlockSpec((1, D), lambda i: (0, 0)),
            pl.BlockSpec((D, D), lambda i: (0, 0)),
            pl.BlockSpec((1, D), lambda i: (0, 0)),
        ],
        out_specs=pl.BlockSpec((NBLK, D), lambda i: (i, 0)),
        out_shape=jax.ShapeDtypeStruct((n, D), jnp.float32),
    )(x, g[None, :], b[None, :], w1, b1[None, :], w2, b2[None, :])


def _pre_body(r_ref, l_ref, wl_ref, bl_ref, wr_ref, a_ref, b_ref):
    a_ref[...] = r_ref[...] @ wl_ref[...] + bl_ref[...]
    b_ref[...] = l_ref[...] @ wr_ref[...]


def _pre(right, left, wl, bl2, wr):
    n = right.shape[0]
    return pl.pallas_call(
        _pre_body,
        grid=(n // NBLK,),
        in_specs=[
            pl.BlockSpec((NBLK, D), lambda i: (i, 0)),
            pl.BlockSpec((NBLK, D), lambda i: (i, 0)),
            pl.BlockSpec((D, D), lambda i: (0, 0)),
            pl.BlockSpec((1, D), lambda i: (0, 0)),
            pl.BlockSpec((D, D), lambda i: (0, 0)),
        ],
        out_specs=[
            pl.BlockSpec((NBLK, D), lambda i: (i, 0)),
            pl.BlockSpec((NBLK, D), lambda i: (i, 0)),
        ],
        out_shape=[
            jax.ShapeDtypeStruct((n, D), jnp.float32),
            jax.ShapeDtypeStruct((n, D), jnp.float32),
        ],
    )(right, left, wl, bl2[None, :], wr)


def _edge_body(ma_ref, mb_ref, g_ref, b_ref, o_ref):
    m = ma_ref[...] + mb_ref[...]
    r = lax.broadcasted_iota(jnp.int32, (2 * D, 2 * D), 0)
    cc = lax.broadcasted_iota(jnp.int32, (2 * D, 2 * D), 1)
    j2 = jnp.where((r // D) == (cc // D), 1.0 / D, 0.0)
    def _dot2(x):
        xh = x.astype(jnp.bfloat16).astype(jnp.float32)
        return jax.lax.dot(xh, j2) + jax.lax.dot(x - xh, j2)

    mu = _dot2(m)
    cen = m - mu
    var = _dot2(cen * cen)
    o_ref[...] = jnp.maximum(
        cen * lax.rsqrt(var + EPS) * g_ref[...] + b_ref[...], 0.0)


def _edge(ma, mb, g, b):
    e = ma.shape[0]
    e2 = e // 2
    ma2 = ma.reshape(e2, 2 * D)
    mb2 = mb.reshape(e2, 2 * D)
    g2 = jnp.concatenate([g, g])[None, :]
    b2 = jnp.concatenate([b, b])[None, :]
    u2 = pl.pallas_call(
        _edge_body,
        grid=(e2 // EBLK,),
        in_specs=[
            pl.BlockSpec((EBLK, 2 * D), lambda i: (i, 0)),
            pl.BlockSpec((EBLK, 2 * D), lambda i: (i, 0)),
            pl.BlockSpec((1, 2 * D), lambda i: (0, 0)),
            pl.BlockSpec((1, 2 * D), lambda i: (0, 0)),
        ],
        out_specs=pl.BlockSpec((EBLK, 2 * D), lambda i: (i, 0)),
        out_shape=jax.ShapeDtypeStruct((e2, 2 * D), jnp.float32),
    )(ma2, mb2, g2, b2)
    return u2.reshape(e, D)


def _post_body(ulo_ref, uhi_ref, deg_ref, r_ref, wf_ref, bf_ref, lg_ref,
               lb_ref, wo1_ref, bo1_ref, wo2_ref, bo2_ref, o_ref):
    u = jnp.concatenate([ulo_ref[0], uhi_ref[0]], axis=-1)
    agg = u @ wf_ref[...] + deg_ref[...][:, :1] * bf_ref[...]
    mu = jnp.mean(agg, axis=-1, keepdims=True)
    var = jnp.mean((agg - mu) ** 2, axis=-1, keepdims=True)
    aggn = (agg - mu) * lax.rsqrt(var + EPS) * lg_ref[...] + lb_ref[...]
    cat = jnp.concatenate([aggn, r_ref[...]], axis=-1)
    h = jnp.maximum(cat @ wo1_ref[...] + bo1_ref[...], 0.0)
    o_ref[...] = h @ wo2_ref[...] + bo2_ref[...]


def _post(u2, deg, right, wf, bf, lg, lb, wo1, bo1, wo2, bo2):
    n = right.shape[0]
    return pl.pallas_call(
        _post_body,
        grid=(n // NBLK,),
        in_specs=[
            pl.BlockSpec((1, NBLK, H), lambda i: (0, i, 0)),
            pl.BlockSpec((1, NBLK, H), lambda i: (1, i, 0)),
            pl.BlockSpec((NBLK, H), lambda i: (i, 0)),
            pl.BlockSpec((NBLK, D), lambda i: (i, 0)),
            pl.BlockSpec((D, D), lambda i: (0, 0)),
            pl.BlockSpec((1, D), lambda i: (0, 0)),
            pl.BlockSpec((1, D), lambda i: (0, 0)),
            pl.BlockSpec((1, D), lambda i: (0, 0)),
            pl.BlockSpec((2 * D, D), lambda i: (0, 0)),
            pl.BlockSpec((1, D), lambda i: (0, 0)),
            pl.BlockSpec((D, D), lambda i: (0, 0)),
            pl.BlockSpec((1, D), lambda i: (0, 0)),
        ],
        out_specs=pl.BlockSpec((NBLK, D), lambda i: (i, 0)),
        out_shape=jax.ShapeDtypeStruct((n, D), jnp.float32),
    )(u2, u2, deg, right, wf, bf[None, :], lg[None, :], lb[None, :],
      wo1, bo1[None, :], wo2, bo2[None, :])


def _head_body(x_ref, w1_ref, b1_ref, w2_ref, o_ref):
    h = jnp.maximum(x_ref[...] @ w1_ref[...] + b1_ref[...], 0.0)
    o_ref[...] = h @ w2_ref[...]


def _head(x, w1, b1, w2p):
    n = x.shape[0]
    return pl.pallas_call(
        _head_body,
        grid=(n // NBLK,),
        in_specs=[
            pl.BlockSpec((NBLK, D), lambda i: (i, 0)),
            pl.BlockSpec((D, D), lambda i: (0, 0)),
            pl.BlockSpec((1, D), lambda i: (0, 0)),
            pl.BlockSpec((D, 128), lambda i: (0, 0)),
        ],
        out_specs=pl.BlockSpec((NBLK, 128), lambda i: (i, 0)),
        out_shape=jax.ShapeDtypeStruct((n, 128), jnp.float32),
    )(x, w1, b1[None, :], w2p)


def _make_gather(npad, epad):
    cpt = epad // (NW * CHUNK)

    @functools.partial(
        pl.kernel,
        mesh=_SC_MESH,
        compiler_params=_SC_PARAMS,
        out_type=[
            jax.ShapeDtypeStruct((epad, D), jnp.float32),
            jax.ShapeDtypeStruct((epad, D), jnp.float32),
        ],
        scratch_types=[
            pltpu.VMEM((cpt, CHUNK), jnp.int32),
            pltpu.VMEM((cpt, CHUNK), jnp.int32),
            pltpu.VMEM((CHUNK, D), jnp.float32),
            pltpu.VMEM((CHUNK, D), jnp.float32),
            pltpu.VMEM((CHUNK, D), jnp.float32),
            pltpu.VMEM((CHUNK, D), jnp.float32),
            pltpu.SemaphoreType.DMA,
            pltpu.SemaphoreType.DMA,
            pltpu.SemaphoreType.DMA,
            pltpu.SemaphoreType.DMA,
            pltpu.SemaphoreType.DMA,
            pltpu.SemaphoreType.DMA,
            pltpu.SemaphoreType.DMA,
            pltpu.SemaphoreType.DMA,
        ],
    )
    def gather(a_hbm, b_hbm, dst2_hbm, src2_hbm, ma_hbm, mb_hbm,
               idxd, idxs, a0, a1, b0, b1,
               gsa0, gsa1, gsb0, gsb1, ssa0, ssa1, ssb0, ssb1):
        wid = lax.axis_index("s") * 2 + lax.axis_index("c")
        cbase = wid * cpt
        pltpu.sync_copy(dst2_hbm.at[pl.ds(cbase, cpt)], idxd)
        pltpu.sync_copy(src2_hbm.at[pl.ds(cbase, cpt)], idxs)

        abufs = (a0, a1)
        bbufs = (b0, b1)
        gsa = (gsa0, gsa1)
        gsb = (gsb0, gsb1)
        ssa = (ssa0, ssa1)
        ssb = (ssb0, ssb1)

        def wait_gather(buf, sem):
            pltpu.make_async_copy(a_hbm.at[pl.ds(0, CHUNK)], buf, sem).wait()

        def start_gather(tbl, t, buf, sem):
            pltpu.async_copy(tbl.at[idxd.at[t] if tbl is a_hbm else idxs.at[t]],
                             buf, sem)

        for par in range(2):
            pltpu.async_copy(a_hbm.at[idxd.at[par]], abufs[par], gsa[par])
            pltpu.async_copy(b_hbm.at[idxs.at[par]], bbufs[par], gsb[par])

        def body(tt, carry):
            sts = []
            for par in range(2):
                t = 2 * tt + par
                rbase = (cbase + t) * CHUNK
                wait_gather(abufs[par], gsa[par])
                sts.append(pltpu.async_copy(abufs[par], ma_hbm.at[pl.ds(rbase, CHUNK)], ssa[par]))
                wait_gather(bbufs[par], gsb[par])
                sts.append(pltpu.async_copy(bbufs[par], mb_hbm.at[pl.ds(rbase, CHUNK)], ssb[par]))
            for par in range(2):
                t = 2 * tt + par
                sts[2 * par].wait()
                pltpu.async_copy(a_hbm.at[idxd.at[t + 2]], abufs[par], gsa[par])
                sts[2 * par + 1].wait()
                pltpu.async_copy(b_hbm.at[idxs.at[t + 2]], bbufs[par], gsb[par])
            return carry

        lax.fori_loop(0, cpt // 2 - 1, body, 0)

        sts = []
        for par in range(2):
            t = cpt - 2 + par
            rbase = (cbase + t) * CHUNK
            wait_gather(abufs[par], gsa[par])
            sts.append(pltpu.async_copy(abufs[par], ma_hbm.at[pl.ds(rbase, CHUNK)], ssa[par]))
            wait_gather(bbufs[par], gsb[par])
            sts.append(pltpu.async_copy(bbufs[par], mb_hbm.at[pl.ds(rbase, CHUNK)], ssb[par]))
        for st in sts:
            st.wait()

    return gather


def _make_scatter(npad, epad):
    SCH = 128
    cps = epad // (16 * SCH)
    rows = npad // 16

    @functools.partial(
        pl.kernel,
        mesh=_SC_MESH,
        compiler_params=_SC_PARAMS,
        out_type=jax.ShapeDtypeStruct((2, npad, H), jnp.float32),
        scratch_types=[
            pltpu.VMEM((1, SCH), jnp.int32),
            pltpu.VMEM((1, SCH), jnp.int32),
            pltpu.VMEM((SCH, H), jnp.float32),
            pltpu.VMEM((SCH, H), jnp.float32),
            pltpu.VMEM_SHARED((npad, H), jnp.float32),
            pltpu.SemaphoreType.DMA,
            pltpu.SemaphoreType.DMA,
            pltpu.SemaphoreType.DMA,
            pltpu.SemaphoreType.DMA,
            pltpu.SemaphoreType.DMA,
            pltpu.SemaphoreType.DMA,
        ],
    )
    def scatter(u2_hbm, dst2_hbm, zeros_hbm, out_hbm,
                i0, i1, u0, u1, acc, li0, li1, lu0, lu1, sc0, sc1):
        c = lax.axis_index("c")
        s = lax.axis_index("s")
        pltpu.sync_copy(zeros_hbm.at[pl.ds(s * rows, rows)],
                        acc.at[pl.ds(s * rows, rows)])
        plsc.subcore_barrier()

        ibufs = (i0, i1)
        ubufs = (u0, u1)
        lisem = (li0, li1)
        lusem = (lu0, lu1)
        ssem = (sc0, sc1)

        def start_loads(t, par):
            pltpu.async_copy(dst2_hbm.at[pl.ds(s * cps + t, 1)], ibufs[par], lisem[par])
            pltpu.async_copy(u2_hbm.at[pl.ds((s * cps + t) * SCH, SCH), pl.ds(c * H, H)],
                             ubufs[par], lusem[par])

        def wait_loads(par):
            pltpu.make_async_copy(dst2_hbm.at[pl.ds(0, 1)], ibufs[par], lisem[par]).wait()
            pltpu.make_async_copy(u2_hbm.at[pl.ds(0, SCH), pl.ds(0, H)], ubufs[par], lusem[par]).wait()

        for par in range(2):
            start_loads(par, par)

        def body(tt, carry):
            scs = []
            for par in range(2):
                wait_loads(par)
                scs.append(pltpu.async_copy(ubufs[par], acc.at[ibufs[par].at[0]],
                                            ssem[par], add=True))
            for par in range(2):
                t = 2 * tt + par
                scs[par].wait()
                start_loads(t + 2, par)
            return carry

        lax.fori_loop(0, cps // 2 - 1, body, 0)

        scs = []
        for par in range(2):
            wait_loads(par)
            scs.append(pltpu.async_copy(ubufs[par], acc.at[ibufs[par].at[0]],
                                        ssem[par], add=True))
        for sc in scs:
            sc.wait()

        plsc.subcore_barrier()
        pltpu.sync_copy(acc.at[pl.ds(s * rows, rows)],
                        out_hbm.at[c, pl.ds(s * rows, rows)])

    return scatter


def _make_deg(npad, epad):
    cps = epad // (16 * CHUNK)
    rows = npad // 16

    @functools.partial(
        pl.kernel,
        mesh=_SC_MESH,
        compiler_params=_SC_PARAMS,
        out_type=jax.ShapeDtypeStruct((2, npad, H), jnp.float32),
        scratch_types=[
            pltpu.VMEM((1, CHUNK), jnp.int32),
            pltpu.VMEM((1, CHUNK), jnp.int32),
            pltpu.VMEM((CHUNK, H), jnp.float32),
            pltpu.VMEM_SHARED((npad, H), jnp.float32),
            pltpu.SemaphoreType.DMA,
            pltpu.SemaphoreType.DMA,
            pltpu.SemaphoreType.DMA,
            pltpu.SemaphoreType.DMA,
        ],
    )
    def deg(ei2_hbm, ones_hbm, zeros_hbm, out_hbm,
            i0, i1, ones_v, acc, li0, li1, sc0, sc1):
        c = lax.axis_index("c")
        s = lax.axis_index("s")
        pltpu.sync_copy(ones_hbm, ones_v)
        pltpu.sync_copy(zeros_hbm.at[pl.ds(s * rows, rows)],
                        acc.at[pl.ds(s * rows, rows)])
        plsc.subcore_barrier()

        ibufs = (i0, i1)
        lisem = (li0, li1)
        ssem = (sc0, sc1)

        def start_load(t, par):
            pltpu.async_copy(ei2_hbm.at[c, pl.ds(s * cps + t, 1)], ibufs[par], lisem[par])

        def wait_load(par):
            pltpu.make_async_copy(ei2_hbm.at[0, pl.ds(0, 1)], ibufs[par], lisem[par]).wait()

        for par in range(2):
            start_load(par, par)

        def body(tt, carry):
            scs = []
            for par in range(2):
                wait_load(par)
                scs.append(pltpu.async_copy(ones_v, acc.at[ibufs[par].at[0]],
                                            ssem[par], add=True))
            for par in range(2):
                t = 2 * tt + par
                scs[par].wait()
                start_load(t + 2, par)
            return carry

        lax.fori_loop(0, cps // 2 - 1, body, 0)

        scs = []
        for par in range(2):
            wait_load(par)
            scs.append(pltpu.async_copy(ones_v, acc.at[ibufs[par].at[0]],
                                        ssem[par], add=True))
        for sc in scs:
            sc.wait()

        plsc.subcore_barrier()
        pltpu.sync_copy(acc.at[pl.ds(s * rows, rows)],
                        out_hbm.at[c, pl.ds(s * rows, rows)])

    return deg


def _conv(left, right, dst, src, dst64, u2_zeros, p, i, deg, gather_fn, scatter_fn):
    econst = p['ee_ln_b'][0] * p['We'][i][0]
    a, b = _pre(right, left, p['Wl'][i], p['bl'][i] + econst, p['Wr'][i])
    ma, mb = gather_fn(a, b, dst, src)
    u2 = _edge(ma, mb, p['lnf_g'][i], p['lnf_b'][i])
    u_seg = scatter_fn(u2, dst, u2_zeros)
    return _post(u_seg, deg, right, p['Wf'][i], p['bf'][i], p['lnp_g'][i],
                 p['lnp_b'][i], p['Wo1'][i], p['bo1'][i], p['Wo2'][i], p['bo2'][i])


def kernel(constraint_features, edge_indices, edge_features, variable_features, params):
    p = params
    n = variable_features.shape[0]
    e = edge_indices.shape[1]
    npad = ((n + NBLK - 1) // NBLK) * NBLK
    epad = ((e + NW * CHUNK - 1) // (NW * CHUNK)) * (NW * CHUNK)

    cfp = jnp.pad(constraint_features, ((0, npad - n), (0, 0)))
    vfp = jnp.pad(variable_features, ((0, npad - n), (0, 0)))
    eip = jnp.pad(edge_indices, ((0, 0), (0, epad - e)), constant_values=n)
    eip_r = eip.reshape(2, epad // CHUNK, CHUNK)
    eip_r64 = eip.reshape(2, epad // 64, 64)
    dst_c, dst_v = eip_r[0], eip_r[1]
    dst_c64, dst_v64 = eip_r64[0], eip_r64[1]
    acc_zeros = jnp.zeros((npad, H), jnp.float32)
    ones_chunk = jnp.ones((CHUNK, H), jnp.float32)

    gather_fn = _make_gather(npad, epad)
    scatter_fn = _make_scatter(npad, epad)
    deg_fn = _make_deg(npad, epad)

    c = _embed(cfp, p['ce_ln_g'], p['ce_ln_b'], p['ce_w1'], p['ce_b1'],
               p['ce_w2'], p['ce_b2'])
    v = _embed(vfp, p['ve_ln_g'], p['ve_ln_b'], p['ve_w1'], p['ve_b1'],
               p['ve_w2'], p['ve_b2'])

    degs = deg_fn(eip_r, ones_chunk, acc_zeros)
    deg_c, deg_v = degs[0], degs[1]

    for l in range(2):
        c = _conv(v, c, dst_c, dst_v, dst_c64, acc_zeros, p, 2 * l, deg_c,
                  gather_fn, scatter_fn)
        v = _conv(c, v, dst_v, dst_c, dst_v64, acc_zeros, p, 2 * l + 1, deg_v,
                  gather_fn, scatter_fn)

    w2p = jnp.pad(p['out_w2'], ((0, 0), (0, 128 - p['out_w2'].shape[1])))
    out = _head(v, p['out_w1'], p['out_b1'], w2p)
    return out[:n, : p['out_w2'].shape[1]]


# speedup vs baseline: 3.0973x; 1.0002x over previous
import functools

import jax
import jax.numpy as jnp
from jax import lax
from jax.experimental import pallas as pl
from jax.experimental.pallas import tpu as pltpu
from jax.experimental.pallas import tpu_sc as plsc

D = 64
H = 32
NBLK = 1024
EBLK = 4096
CHUNK = 128
NW = 32
EPS = 1e-5

_SC_MESH = plsc.VectorSubcoreMesh(core_axis_name="c", subcore_axis_name="s")
_SC_PARAMS = pltpu.CompilerParams(use_tc_tiling_on_sc=False)


def _embed_body(x_ref, g_ref, b_ref, w1_ref, b1_ref, w2_ref, b2_ref, o_ref):
    x = x_ref[...]
    m = jnp.mean(x, axis=-1, keepdims=True)
    v = jnp.mean((x - m) ** 2, axis=-1, keepdims=True)
    xn = (x - m) * lax.rsqrt(v + EPS) * g_ref[...] + b_ref[...]
    h = jnp.maximum(xn @ w1_ref[...] + b1_ref[...], 0.0)
    o_ref[...] = jnp.maximum(h @ w2_ref[...] + b2_ref[...], 0.0)


def _embed(x, g, b, w1, b1, w2, b2):
    n, f = x.shape
    return pl.pallas_call(
        _embed_body,
        grid=(n // NBLK,),
        in_specs=[
            pl.BlockSpec((NBLK, f), lambda i: (i, 0)),
            pl.BlockSpec((1, f), lambda i: (0, 0)),
            pl.BlockSpec((1, f), lambda i: (0, 0)),
            pl.BlockSpec((f, D), lambda i: (0, 0)),
            pl.BlockSpec((1, D), lambda i: (0, 0)),
            pl.BlockSpec((D, D), lambda i: (0, 0)),
            pl.BlockSpec((1, D), lambda i: (0, 0)),
        ],
        out_specs=pl.BlockSpec((NBLK, D), lambda i: (i, 0)),
        out_shape=jax.ShapeDtypeStruct((n, D), jnp.float32),
    )(x, g[None, :], b[None, :], w1, b1[None, :], w2, b2[None, :])


def _pre_body(r_ref, l_ref, wl_ref, bl_ref, wr_ref, a_ref, b_ref):
    a_ref[...] = r_ref[...] @ wl_ref[...] + bl_ref[...]
    b_ref[...] = l_ref[...] @ wr_ref[...]


def _pre(right, left, wl, bl2, wr):
    n = right.shape[0]
    return pl.pallas_call(
        _pre_body,
        grid=(n // NBLK,),
        in_specs=[
            pl.BlockSpec((NBLK, D), lambda i: (i, 0)),
            pl.BlockSpec((NBLK, D), lambda i: (i, 0)),
            pl.BlockSpec((D, D), lambda i: (0, 0)),
            pl.BlockSpec((1, D), lambda i: (0, 0)),
            pl.BlockSpec((D, D), lambda i: (0, 0)),
        ],
        out_specs=[
            pl.BlockSpec((NBLK, D), lambda i: (i, 0)),
            pl.BlockSpec((NBLK, D), lambda i: (i, 0)),
        ],
        out_shape=[
            jax.ShapeDtypeStruct((n, D), jnp.float32),
            jax.ShapeDtypeStruct((n, D), jnp.float32),
        ],
    )(right, left, wl, bl2[None, :], wr)


def _edge_body(ma_ref, mb_ref, g_ref, b_ref, o_ref):
    m = ma_ref[...] + mb_ref[...]
    r = lax.broadcasted_iota(jnp.int32, (2 * D, 2 * D), 0)
    cc = lax.broadcasted_iota(jnp.int32, (2 * D, 2 * D), 1)
    j2 = jnp.where((r // D) == (cc // D), 1.0 / D, 0.0)
    def _dot2(x):
        xh = x.astype(jnp.bfloat16).astype(jnp.float32)
        return jax.lax.dot(xh, j2) + jax.lax.dot(x - xh, j2)

    mu = _dot2(m)
    cen = m - mu
    var = _dot2(cen * cen)
    o_ref[...] = jnp.maximum(
        cen * lax.rsqrt(var + EPS) * g_ref[...] + b_ref[...], 0.0)


def _edge(ma, mb, g, b):
    e = ma.shape[0]
    e2 = e // 2
    ma2 = ma.reshape(e2, 2 * D)
    mb2 = mb.reshape(e2, 2 * D)
    g2 = jnp.concatenate([g, g])[None, :]
    b2 = jnp.concatenate([b, b])[None, :]
    u2 = pl.pallas_call(
        _edge_body,
        grid=(e2 // EBLK,),
        in_specs=[
            pl.BlockSpec((EBLK, 2 * D), lambda i: (i, 0)),
            pl.BlockSpec((EBLK, 2 * D), lambda i: (i, 0)),
            pl.BlockSpec((1, 2 * D), lambda i: (0, 0)),
            pl.BlockSpec((1, 2 * D), lambda i: (0, 0)),
        ],
        out_specs=pl.BlockSpec((EBLK, 2 * D), lambda i: (i, 0)),
        out_shape=jax.ShapeDtypeStruct((e2, 2 * D), jnp.float32),
    )(ma2, mb2, g2, b2)
    return u2.reshape(e, D)


def _post_body(ulo_ref, uhi_ref, deg_ref, r_ref, wf_ref, bf_ref, lg_ref,
               lb_ref, wo1_ref, bo1_ref, wo2_ref, bo2_ref, o_ref):
    u = jnp.concatenate([ulo_ref[0], uhi_ref[0]], axis=-1)
    agg = u @ wf_ref[...] + deg_ref[...][:, :1] * bf_ref[...]
    mu = jnp.mean(agg, axis=-1, keepdims=True)
    var = jnp.mean((agg - mu) ** 2, axis=-1, keepdims=True)
    aggn = (agg - mu) * lax.rsqrt(var + EPS) * lg_ref[...] + lb_ref[...]
    cat = jnp.concatenate([aggn, r_ref[...]], axis=-1)
    h = jnp.maximum(cat @ wo1_ref[...] + bo1_ref[...], 0.0)
    o_ref[...] = h @ wo2_ref[...] + bo2_ref[...]


def _post(u2, deg, right, wf, bf, lg, lb, wo1, bo1, wo2, bo2):
    n = right.shape[0]
    return pl.pallas_call(
        _post_body,
        grid=(n // NBLK,),
        in_specs=[
            pl.BlockSpec((1, NBLK, H), lambda i: (0, i, 0)),
            pl.BlockSpec((1, NBLK, H), lambda i: (1, i, 0)),
            pl.BlockSpec((NBLK, H), lambda i: (i, 0)),
            pl.BlockSpec((NBLK, D), lambda i: (i, 0)),
            pl.BlockSpec((D, D), lambda i: (0, 0)),
            pl.BlockSpec((1, D), lambda i: (0, 0)),
            pl.BlockSpec((1, D), lambda i: (0, 0)),
            pl.BlockSpec((1, D), lambda i: (0, 0)),
            pl.BlockSpec((2 * D, D), lambda i: (0, 0)),
            pl.BlockSpec((1, D), lambda i: (0, 0)),
            pl.BlockSpec((D, D), lambda i: (0, 0)),
            pl.BlockSpec((1, D), lambda i: (0, 0)),
        ],
        out_specs=pl.BlockSpec((NBLK, D), lambda i: (i, 0)),
        out_shape=jax.ShapeDtypeStruct((n, D), jnp.float32),
    )(u2, u2, deg, right, wf, bf[None, :], lg[None, :], lb[None, :],
      wo1, bo1[None, :], wo2, bo2[None, :])


def _head_body(x_ref, w1_ref, b1_ref, w2_ref, o_ref):
    h = jnp.maximum(x_ref[...] @ w1_ref[...] + b1_ref[...], 0.0)
    o_ref[...] = h @ w2_ref[...]


def _head(x, w1, b1, w2p):
    n = x.shape[0]
    return pl.pallas_call(
        _head_body,
        grid=(n // NBLK,),
        in_specs=[
            pl.BlockSpec((NBLK, D), lambda i: (i, 0)),
            pl.BlockSpec((D, D), lambda i: (0, 0)),
            pl.BlockSpec((1, D), lambda i: (0, 0)),
            pl.BlockSpec((D, 128), lambda i: (0, 0)),
        ],
        out_specs=pl.BlockSpec((NBLK, 128), lambda i: (i, 0)),
        out_shape=jax.ShapeDtypeStruct((n, 128), jnp.float32),
    )(x, w1, b1[None, :], w2p)


def _make_gather(npad, epad):
    cpt = epad // (NW * CHUNK)

    @functools.partial(
        pl.kernel,
        mesh=_SC_MESH,
        compiler_params=_SC_PARAMS,
        out_type=[
            jax.ShapeDtypeStruct((epad, D), jnp.float32),
            jax.ShapeDtypeStruct((epad, D), jnp.float32),
        ],
        scratch_types=[
            pltpu.VMEM((cpt, CHUNK), jnp.int32),
            pltpu.VMEM((cpt, CHUNK), jnp.int32),
            pltpu.VMEM((CHUNK, D), jnp.float32),
            pltpu.VMEM((CHUNK, D), jnp.float32),
            pltpu.VMEM((CHUNK, D), jnp.float32),
            pltpu.VMEM((CHUNK, D), jnp.float32),
            pltpu.SemaphoreType.DMA,
            pltpu.SemaphoreType.DMA,
            pltpu.SemaphoreType.DMA,
            pltpu.SemaphoreType.DMA,
            pltpu.SemaphoreType.DMA,
            pltpu.SemaphoreType.DMA,
            pltpu.SemaphoreType.DMA,
            pltpu.SemaphoreType.DMA,
        ],
    )
    def gather(a_hbm, b_hbm, dst2_hbm, src2_hbm, ma_hbm, mb_hbm,
               idxd, idxs, a0, a1, b0, b1,
               gsa0, gsa1, gsb0, gsb1, ssa0, ssa1, ssb0, ssb1):
        wid = lax.axis_index("s") * 2 + lax.axis_index("c")
        cbase = wid * cpt
        pltpu.sync_copy(dst2_hbm.at[pl.ds(cbase, cpt)], idxd)
        pltpu.sync_copy(src2_hbm.at[pl.ds(cbase, cpt)], idxs)

        abufs = (a0, a1)
        bbufs = (b0, b1)
        gsa = (gsa0, gsa1)
        gsb = (gsb0, gsb1)
        ssa = (ssa0, ssa1)
        ssb = (ssb0, ssb1)

        def wait_gather(buf, sem):
            pltpu.make_async_copy(a_hbm.at[pl.ds(0, CHUNK)], buf, sem).wait()

        for par in range(2):
            pltpu.async_copy(a_hbm.at[idxd.at[par]], abufs[par], gsa[par])
            pltpu.async_copy(b_hbm.at[idxs.at[par]], bbufs[par], gsb[par])

        def body(tt, carry):
            sts = []
            for par in range(2):
                t = 2 * tt + par
                rbase = (cbase + t) * CHUNK
                wait_gather(abufs[par], gsa[par])
                sts.append(pltpu.async_copy(abufs[par], ma_hbm.at[pl.ds(rbase, CHUNK)], ssa[par]))
                wait_gather(bbufs[par], gsb[par])
                sts.append(pltpu.async_copy(bbufs[par], mb_hbm.at[pl.ds(rbase, CHUNK)], ssb[par]))
            for par in range(2):
                t = 2 * tt + par
                sts[2 * par].wait()
                pltpu.async_copy(a_hbm.at[idxd.at[t + 2]], abufs[par], gsa[par])
                sts[2 * par + 1].wait()
                pltpu.async_copy(b_hbm.at[idxs.at[t + 2]], bbufs[par], gsb[par])
            return carry

        lax.fori_loop(0, cpt // 2 - 1, body, 0)

        sts = []
        for par in range(2):
            t = cpt - 2 + par
            rbase = (cbase + t) * CHUNK
            wait_gather(abufs[par], gsa[par])
            sts.append(pltpu.async_copy(abufs[par], ma_hbm.at[pl.ds(rbase, CHUNK)], ssa[par]))
            wait_gather(bbufs[par], gsb[par])
            sts.append(pltpu.async_copy(bbufs[par], mb_hbm.at[pl.ds(rbase, CHUNK)], ssb[par]))
        for st in sts:
            st.wait()

    return gather


def _make_scatter(npad, epad):
    SCH = 128
    cps = epad // (16 * SCH)
    rows = npad // 16

    @functools.partial(
        pl.kernel,
        mesh=_SC_MESH,
        compiler_params=_SC_PARAMS,
        out_type=jax.ShapeDtypeStruct((2, npad, H), jnp.float32),
        scratch_types=[
            pltpu.VMEM((1, SCH), jnp.int32),
            pltpu.VMEM((1, SCH), jnp.int32),
            pltpu.VMEM((SCH, H), jnp.float32),
            pltpu.VMEM((SCH, H), jnp.float32),
            pltpu.VMEM_SHARED((npad, H), jnp.float32),
            pltpu.SemaphoreType.DMA,
            pltpu.SemaphoreType.DMA,
            pltpu.SemaphoreType.DMA,
            pltpu.SemaphoreType.DMA,
            pltpu.SemaphoreType.DMA,
            pltpu.SemaphoreType.DMA,
        ],
    )
    def scatter(u2_hbm, dst2_hbm, zeros_hbm, out_hbm,
                i0, i1, u0, u1, acc, li0, li1, lu0, lu1, sc0, sc1):
        c = lax.axis_index("c")
        s = lax.axis_index("s")
        pltpu.sync_copy(zeros_hbm.at[pl.ds(s * rows, rows)],
                        acc.at[pl.ds(s * rows, rows)])
        plsc.subcore_barrier()

        ibufs = (i0, i1)
        ubufs = (u0, u1)
        lisem = (li0, li1)
        lusem = (lu0, lu1)
        ssem = (sc0, sc1)

        def start_loads(t, par):
            pltpu.async_copy(dst2_hbm.at[pl.ds(s * cps + t, 1)], ibufs[par], lisem[par])
            pltpu.async_copy(u2_hbm.at[pl.ds((s * cps + t) * SCH, SCH), pl.ds(c * H, H)],
                             ubufs[par], lusem[par])

        def wait_loads(par):
            pltpu.make_async_copy(dst2_hbm.at[pl.ds(0, 1)], ibufs[par], lisem[par]).wait()
            pltpu.make_async_copy(u2_hbm.at[pl.ds(0, SCH), pl.ds(0, H)], ubufs[par], lusem[par]).wait()

        for par in range(2):
            start_loads(par, par)

        def body(tt, carry):
            scs = []
            for par in range(2):
                wait_loads(par)
                scs.append(pltpu.async_copy(ubufs[par], acc.at[ibufs[par].at[0]],
                                            ssem[par], add=True))
            for par in range(2):
                t = 2 * tt + par
                scs[par].wait()
                start_loads(t + 2, par)
            return carry

        lax.fori_loop(0, cps // 2 - 1, body, 0)

        scs = []
        for par in range(2):
            wait_loads(par)
            scs.append(pltpu.async_copy(ubufs[par], acc.at[ibufs[par].at[0]],
                                        ssem[par], add=True))
        for sc in scs:
            sc.wait()

        plsc.subcore_barrier()
        pltpu.sync_copy(acc.at[pl.ds(s * rows, rows)],
                        out_hbm.at[c, pl.ds(s * rows, rows)])

    return scatter


def _make_deg(npad, epad):
    cps = epad // (16 * CHUNK)
    rows = npad // 16

    @functools.partial(
        pl.kernel,
        mesh=_SC_MESH,
        compiler_params=_SC_PARAMS,
        out_type=jax.ShapeDtypeStruct((2, npad, H), jnp.float32),
        scratch_types=[
            pltpu.VMEM((1, CHUNK), jnp.int32),
            pltpu.VMEM((1, CHUNK), jnp.int32),
            pltpu.VMEM((CHUNK, H), jnp.float32),
            pltpu.VMEM_SHARED((npad, H), jnp.float32),
            pltpu.SemaphoreType.DMA,
            pltpu.SemaphoreType.DMA,
            pltpu.SemaphoreType.DMA,
            pltpu.SemaphoreType.DMA,
        ],
    )
    def deg(ei2_hbm, ones_hbm, zeros_hbm, out_hbm,
            i0, i1, ones_v, acc, li0, li1, sc0, sc1):
        c = lax.axis_index("c")
        s = lax.axis_index("s")
        pltpu.sync_copy(ones_hbm, ones_v)
        pltpu.sync_copy(zeros_hbm.at[pl.ds(s * rows, rows)],
                        acc.at[pl.ds(s * rows, rows)])
        plsc.subcore_barrier()

        ibufs = (i0, i1)
        lisem = (li0, li1)
        ssem = (sc0, sc1)

        def start_load(t, par):
            pltpu.async_copy(ei2_hbm.at[c, pl.ds(s * cps + t, 1)], ibufs[par], lisem[par])

        def wait_load(par):
            pltpu.make_async_copy(ei2_hbm.at[0, pl.ds(0, 1)], ibufs[par], lisem[par]).wait()

        for par in range(2):
            start_load(par, par)

        def body(tt, carry):
            scs = []
            for par in range(2):
                wait_load(par)
                scs.append(pltpu.async_copy(ones_v, acc.at[ibufs[par].at[0]],
                                            ssem[par], add=True))
            for par in range(2):
                t = 2 * tt + par
                scs[par].wait()
                start_load(t + 2, par)
            return carry

        lax.fori_loop(0, cps // 2 - 1, body, 0)

        scs = []
        for par in range(2):
            wait_load(par)
            scs.append(pltpu.async_copy(ones_v, acc.at[ibufs[par].at[0]],
                                        ssem[par], add=True))
        for sc in scs:
            sc.wait()

        plsc.subcore_barrier()
        pltpu.sync_copy(acc.at[pl.ds(s * rows, rows)],
                        out_hbm.at[c, pl.ds(s * rows, rows)])

    return deg


def _conv(left, right, dst, src, u2_zeros, p, i, deg, gather_fn, scatter_fn):
    econst = p['ee_ln_b'][0] * p['We'][i][0]
    a, b = _pre(right, left, p['Wl'][i], p['bl'][i] + econst, p['Wr'][i])
    ma, mb = gather_fn(a, b, dst, src)
    u2 = _edge(ma, mb, p['lnf_g'][i], p['lnf_b'][i])
    u_seg = scatter_fn(u2, dst, u2_zeros)
    return _post(u_seg, deg, right, p['Wf'][i], p['bf'][i], p['lnp_g'][i],
                 p['lnp_b'][i], p['Wo1'][i], p['bo1'][i], p['Wo2'][i], p['bo2'][i])


def kernel(constraint_features, edge_indices, edge_features, variable_features, params):
    p = params
    n = variable_features.shape[0]
    e = edge_indices.shape[1]
    npad = ((n + NBLK - 1) // NBLK) * NBLK
    epad = ((e + NW * CHUNK - 1) // (NW * CHUNK)) * (NW * CHUNK)

    cfp = jnp.pad(constraint_features, ((0, npad - n), (0, 0)))
    vfp = jnp.pad(variable_features, ((0, npad - n), (0, 0)))
    eip = jnp.pad(edge_indices, ((0, 0), (0, epad - e)), constant_values=n)
    eip_r = eip.reshape(2, epad // CHUNK, CHUNK)
    dst_c, dst_v = eip_r[0], eip_r[1]
    acc_zeros = jnp.zeros((npad, H), jnp.float32)
    ones_chunk = jnp.ones((CHUNK, H), jnp.float32)

    gather_fn = _make_gather(npad, epad)
    scatter_fn = _make_scatter(npad, epad)
    deg_fn = _make_deg(npad, epad)

    c = _embed(cfp, p['ce_ln_g'], p['ce_ln_b'], p['ce_w1'], p['ce_b1'],
               p['ce_w2'], p['ce_b2'])
    v = _embed(vfp, p['ve_ln_g'], p['ve_ln_b'], p['ve_w1'], p['ve_b1'],
               p['ve_w2'], p['ve_b2'])

    degs = deg_fn(eip_r, ones_chunk, acc_zeros)
    deg_c, deg_v = degs[0], degs[1]

    for l in range(2):
        c = _conv(v, c, dst_c, dst_v, acc_zeros, p, 2 * l, deg_c,
                  gather_fn, scatter_fn)
        v = _conv(c, v, dst_v, dst_c, acc_zeros, p, 2 * l + 1, deg_v,
                  gather_fn, scatter_fn)

    w2p = jnp.pad(p['out_w2'], ((0, 0), (0, 128 - p['out_w2'].shape[1])))
    out = _head(v, p['out_w1'], p['out_b1'], w2p)
    return out[:n, : p['out_w2'].shape[1]]
